# Initial kernel scaffold; baseline (speedup 1.0000x reference)
import jax
import jax.numpy as jnp
from jax.experimental import pallas as pl


def kernel(feature_indices, feature_values, edge_indices, edge_weights, W1, b1, W2, b2):
    raise NotImplementedError("write your pallas kernel here")


# speedup vs baseline: 4.2092x; 4.2092x over previous
import dataclasses
import functools

import jax
import jax.numpy as jnp
from jax import lax
from jax.experimental import pallas as pl
from jax.experimental.pallas import tpu as pltpu
from jax.experimental.pallas import tpu_sc as plsc

NN = 50000
FF = 128
HH = 128
LL = 64
EE = 800000
ZF = 1000000
AL = 0.1
IT = 10

NSUB = 16
HALF = LL // 2
QROWS = NN // 4
QFLAT = QROWS * FF

F_ROWS = 7824
E_ROWS = 6256
F_CHUNKS = F_ROWS // 16
E_CHUNKS = E_ROWS // 8

BR = 80


def _sc_mesh():
    return plsc.VectorSubcoreMesh(core_axis_name="c", subcore_axis_name="s")


def _sc_params():
    cp = pltpu.CompilerParams()
    if "needs_layout_passes" in pltpu.CompilerParams.__dataclass_fields__:
        cp = dataclasses.replace(cp, needs_layout_passes=False)
    if "use_tc_tiling_on_sc" in pltpu.CompilerParams.__dataclass_fields__:
        cp = dataclasses.replace(cp, use_tc_tiling_on_sc=False)
    return cp


def _densify(frow, fcol, fval):
    @functools.partial(
        pl.kernel,
        out_type=jax.ShapeDtypeStruct((NN * FF,), jnp.float32),
        mesh=_sc_mesh(),
        scratch_types=[
            pltpu.VMEM((16, 128), jnp.int32),
            pltpu.VMEM((16, 128), jnp.int32),
            pltpu.VMEM((16, 128), jnp.float32),
            pltpu.VMEM((10000,), jnp.float32),
            pltpu.VMEM((10000,), jnp.float32),
            pltpu.VMEM_SHARED((QFLAT,), jnp.float32),
            pltpu.SemaphoreType.DMA,
        ],
    )
    def k(frow_hbm, fcol_hbm, fval_hbm, x_hbm, rb, cb, vb, stage, stage2,
          acc, sem):
        c = lax.axis_index("c")
        s = lax.axis_index("s")

        @pl.loop(0, 10000, step=16)
        def _zstage(i):
            stage[pl.ds(i, 16)] = jnp.zeros((16,), jnp.float32)

        @pl.loop(0, 2)
        def _pass(p):
            q = 2 * c + p
            base = q * QROWS

            @pl.loop(0, 10)
            def _z(i):
                pltpu.sync_copy(
                    stage, acc.at[pl.ds(s * 100000 + i * 10000, 10000)])

            plsc.subcore_barrier()

            @pl.loop(s, F_CHUNKS, step=NSUB)
            def _chunk(j):
                roff = j * 16
                pltpu.sync_copy(frow_hbm.at[pl.ds(roff, 16)], rb)
                pltpu.sync_copy(fcol_hbm.at[pl.ds(roff, 16)], cb)
                pltpu.sync_copy(fval_hbm.at[pl.ds(roff, 16)], vb)

                @pl.loop(0, 16)
                def _row(i):
                    @pl.loop(0, 128, step=16)
                    def _lane(l):
                        r = rb[i, pl.ds(l, 16)]
                        cc = cb[i, pl.ds(l, 16)]
                        v = vb[i, pl.ds(l, 16)]
                        rel = r - base
                        ok = (rel >= 0) & (rel < QROWS)
                        rb[i, pl.ds(l, 16)] = jnp.where(ok, rel * FF + cc, 0)
                        vb[i, pl.ds(l, 16)] = jnp.where(
                            ok, v, jnp.zeros_like(v))

                for t in range(16):
                    pltpu.sync_copy(vb.at[t], acc.at[rb.at[t]], add=True)

            plsc.subcore_barrier()

            @pl.loop(0, 10)
            def _out(i):
                off = s * 100000 + i * 10000
                pltpu.sync_copy(acc.at[pl.ds(off, 10000)], stage2)
                pltpu.sync_copy(stage2,
                                x_hbm.at[pl.ds(q * QFLAT + off, 10000)])

    return k(frow, fcol, fval)


def _mlp(x, w1, b1, w2, b2):
    bm = 2000
    grid = (NN // bm,)

    def body(x_ref, w1_ref, b1_ref, w2_ref, b2_ref, z0_ref, z1_ref):
        h = lax.dot_general(
            x_ref[...], w1_ref[...], (((1,), (0,)), ((), ())),
            precision=lax.Precision.HIGHEST,
            preferred_element_type=jnp.float32)
        h = jnp.maximum(h + b1_ref[...], 0.0)
        z = lax.dot_general(
            h, w2_ref[...], (((1,), (0,)), ((), ())),
            precision=lax.Precision.HIGHEST,
            preferred_element_type=jnp.float32) + b2_ref[...]
        z0_ref[...] = z[:, :HALF]
        z1_ref[...] = z[:, HALF:]

    return pl.pallas_call(
        body,
        grid=grid,
        in_specs=[
            pl.BlockSpec((bm, FF), lambda i: (i, 0)),
            pl.BlockSpec((FF, HH), lambda i: (0, 0)),
            pl.BlockSpec((1, HH), lambda i: (0, 0)),
            pl.BlockSpec((HH, LL), lambda i: (0, 0)),
            pl.BlockSpec((1, LL), lambda i: (0, 0)),
        ],
        out_specs=[
            pl.BlockSpec((bm, HALF), lambda i: (i, 0)),
            pl.BlockSpec((bm, HALF), lambda i: (i, 0)),
        ],
        out_shape=[
            jax.ShapeDtypeStruct((NN, HALF), jnp.float32),
            jax.ShapeDtypeStruct((NN, HALF), jnp.float32),
        ],
    )(x, w1, b1, w2, b2)


def _propagate(z0, z1, src, dst, w):
    @functools.partial(
        pl.kernel,
        out_type=(
            jax.ShapeDtypeStruct((NN, HALF), jnp.float32),
            jax.ShapeDtypeStruct((NN, HALF), jnp.float32),
        ),
        mesh=_sc_mesh(),
        compiler_params=_sc_params(),
        scratch_types=[
            pltpu.VMEM((8, 128), jnp.int32),
            pltpu.VMEM((8, 128), jnp.int32),
            pltpu.VMEM((8, 128), jnp.float32),
            pltpu.VMEM((128, HALF), jnp.float32),
            pltpu.VMEM((BR, HALF), jnp.float32),
            pltpu.VMEM((BR, HALF), jnp.float32),
            pltpu.VMEM((BR, HALF), jnp.float32),
            pltpu.VMEM_SHARED((NN, HALF), jnp.float32),
            pltpu.SemaphoreType.DMA,
        ],
    )
    def k(z0_hbm, z1_hbm, src_hbm, dst_hbm, w_hbm, l0_hbm, l1_hbm,
          sb, db, wb, rows, ba, bb, zb, acc, sem):
        c = lax.axis_index("c")
        s = lax.axis_index("s")

        @pl.loop(0, BR)
        def _zzb(r):
            zb[r, pl.ds(0, 16)] = jnp.zeros((16,), jnp.float32)
            zb[r, pl.ds(16, 16)] = jnp.zeros((16,), jnp.float32)

        def work(z_ref, l_ref):
            @pl.loop(s, NN // BR, step=NSUB)
            def _pro(kk):
                r0 = kk * BR
                pltpu.sync_copy(z_ref.at[pl.ds(r0, BR)], ba)
                pltpu.sync_copy(ba, l_ref.at[pl.ds(r0, BR)])
                pltpu.sync_copy(zb, acc.at[pl.ds(r0, BR)])

            plsc.subcore_barrier()

            @pl.loop(0, IT)
            def _iter(it):
                @pl.loop(s, E_CHUNKS, step=NSUB)
                def _chunk(j):
                    roff = j * 8
                    pltpu.sync_copy(src_hbm.at[pl.ds(roff, 8)], sb)
                    pltpu.sync_copy(dst_hbm.at[pl.ds(roff, 8)], db)
                    pltpu.sync_copy(w_hbm.at[pl.ds(roff, 8)], wb)

                    @pl.loop(0, 8)
                    def _sub(t):
                        pltpu.async_copy(
                            l_ref.at[sb.at[t]], rows, sem).wait()
                        tv = jnp.broadcast_to(t, (16,))

                        @pl.loop(0, 128, step=16)
                        def _scale(e0):
                            for i in range(16):
                                ws = plsc.load_gather(
                                    wb,
                                    [tv, jnp.broadcast_to(e0 + i, (16,))])
                                rows[e0 + i, pl.ds(0, 16)] = (
                                    rows[e0 + i, pl.ds(0, 16)] * ws)
                                rows[e0 + i, pl.ds(16, 16)] = (
                                    rows[e0 + i, pl.ds(16, 16)] * ws)

                        pltpu.sync_copy(rows, acc.at[db.at[t]], add=True)

                plsc.subcore_barrier()

                @pl.loop(s, NN // BR, step=NSUB)
                def _blend(kk):
                    r0 = kk * BR
                    pltpu.sync_copy(acc.at[pl.ds(r0, BR)], ba)
                    pltpu.sync_copy(z_ref.at[pl.ds(r0, BR)], bb)

                    @pl.loop(0, BR)
                    def _rowb(r):
                        a0 = ba[r, pl.ds(0, 16)]
                        b0 = bb[r, pl.ds(0, 16)]
                        ba[r, pl.ds(0, 16)] = (1.0 - AL) * a0 + AL * b0
                        a1 = ba[r, pl.ds(16, 16)]
                        b1 = bb[r, pl.ds(16, 16)]
                        ba[r, pl.ds(16, 16)] = (1.0 - AL) * a1 + AL * b1

                    pltpu.sync_copy(ba, l_ref.at[pl.ds(r0, BR)])
                    pltpu.sync_copy(zb, acc.at[pl.ds(r0, BR)])

                plsc.subcore_barrier()

        @pl.when(c == 0)
        def _():
            work(z0_hbm, l0_hbm)

        @pl.when(c == 1)
        def _():
            work(z1_hbm, l1_hbm)

    return k(z0, z1, src, dst, w)


def _log_softmax(l0, l1):
    bm = 2000
    grid = (NN // bm,)

    def body(l0_ref, l1_ref, o_ref):
        x = jnp.concatenate([l0_ref[...], l1_ref[...]], axis=1)
        m = jnp.max(x, axis=1, keepdims=True)
        xm = x - m
        lse = jnp.log(jnp.sum(jnp.exp(xm), axis=1, keepdims=True))
        o_ref[...] = xm - lse

    return pl.pallas_call(
        body,
        grid=grid,
        in_specs=[
            pl.BlockSpec((bm, HALF), lambda i: (i, 0)),
            pl.BlockSpec((bm, HALF), lambda i: (i, 0)),
        ],
        out_specs=pl.BlockSpec((bm, LL), lambda i: (i, 0)),
        out_shape=jax.ShapeDtypeStruct((NN, LL), jnp.float32),
    )(l0, l1)


def kernel(feature_indices, feature_values, edge_indices, edge_weights,
           W1, b1, W2, b2):
    frow = feature_indices[0].astype(jnp.int32)
    fcol = feature_indices[1].astype(jnp.int32)
    fval = feature_values.astype(jnp.float32)
    fpad = F_ROWS * 128 - ZF
    frow = jnp.concatenate([frow, jnp.zeros((fpad,), jnp.int32)]).reshape(
        F_ROWS, 128)
    fcol = jnp.concatenate([fcol, jnp.zeros((fpad,), jnp.int32)]).reshape(
        F_ROWS, 128)
    fval = jnp.concatenate([fval, jnp.zeros((fpad,), jnp.float32)]).reshape(
        F_ROWS, 128)

    src = edge_indices[1].astype(jnp.int32)
    dst = edge_indices[0].astype(jnp.int32)
    wgt = edge_weights.astype(jnp.float32)
    epad = E_ROWS * 128 - EE
    src = jnp.concatenate([src, jnp.zeros((epad,), jnp.int32)]).reshape(
        E_ROWS, 128)
    dst = jnp.concatenate([dst, jnp.zeros((epad,), jnp.int32)]).reshape(
        E_ROWS, 128)
    wgt = jnp.concatenate([wgt, jnp.zeros((epad,), jnp.float32)]).reshape(
        E_ROWS, 128)

    x = _densify(frow, fcol, fval).reshape(NN, FF)
    z0, z1 = _mlp(x, W1, b1.reshape(1, HH), W2, b2.reshape(1, LL))
    l0, l1 = _propagate(z0, z1, src, dst, wgt)
    return _log_softmax(l0, l1)


# speedup vs baseline: 6.0646x; 1.4408x over previous
import dataclasses
import functools

import jax
import jax.numpy as jnp
from jax import lax
from jax.experimental import pallas as pl
from jax.experimental.pallas import tpu as pltpu
from jax.experimental.pallas import tpu_sc as plsc

NN = 50000
FF = 128
HH = 128
LL = 64
EE = 800000
ZF = 1000000
AL = 0.1
IT = 10

NSUB = 16
HALF = LL // 2
QROWS = NN // 4
QFLAT = QROWS * FF

F_ROWS = 7824
E_ROWS = 6400
F_CHUNKS = F_ROWS // 16
E_CHUNKS = E_ROWS // 8
KTILE = E_CHUNKS // NSUB

BR = 80


def _sc_mesh():
    return plsc.VectorSubcoreMesh(core_axis_name="c", subcore_axis_name="s")


def _sc_params():
    cp = pltpu.CompilerParams()
    if "needs_layout_passes" in pltpu.CompilerParams.__dataclass_fields__:
        cp = dataclasses.replace(cp, needs_layout_passes=False)
    if "use_tc_tiling_on_sc" in pltpu.CompilerParams.__dataclass_fields__:
        cp = dataclasses.replace(cp, use_tc_tiling_on_sc=False)
    return cp


def _densify(frow, fcol, fval):
    @functools.partial(
        pl.kernel,
        out_type=jax.ShapeDtypeStruct((NN * FF,), jnp.float32),
        mesh=_sc_mesh(),
        scratch_types=[
            pltpu.VMEM((16, 128), jnp.int32),
            pltpu.VMEM((16, 128), jnp.int32),
            pltpu.VMEM((16, 128), jnp.float32),
            pltpu.VMEM((10000,), jnp.float32),
            pltpu.VMEM((10000,), jnp.float32),
            pltpu.VMEM_SHARED((QFLAT,), jnp.float32),
            pltpu.SemaphoreType.DMA,
        ],
    )
    def k(frow_hbm, fcol_hbm, fval_hbm, x_hbm, rb, cb, vb, stage, stage2,
          acc, sem):
        c = lax.axis_index("c")
        s = lax.axis_index("s")

        @pl.loop(0, 10000, step=16)
        def _zstage(i):
            stage[pl.ds(i, 16)] = jnp.zeros((16,), jnp.float32)

        @pl.loop(0, 2)
        def _pass(p):
            q = 2 * c + p
            base = q * QROWS

            @pl.loop(0, 10)
            def _z(i):
                pltpu.sync_copy(
                    stage, acc.at[pl.ds(s * 100000 + i * 10000, 10000)])

            plsc.subcore_barrier()

            @pl.loop(s, F_CHUNKS, step=NSUB)
            def _chunk(j):
                roff = j * 16
                pltpu.sync_copy(frow_hbm.at[pl.ds(roff, 16)], rb)
                pltpu.sync_copy(fcol_hbm.at[pl.ds(roff, 16)], cb)
                pltpu.sync_copy(fval_hbm.at[pl.ds(roff, 16)], vb)

                @pl.loop(0, 16)
                def _row(i):
                    @pl.loop(0, 128, step=16)
                    def _lane(l):
                        r = rb[i, pl.ds(l, 16)]
                        cc = cb[i, pl.ds(l, 16)]
                        v = vb[i, pl.ds(l, 16)]
                        rel = r - base
                        ok = (rel >= 0) & (rel < QROWS)
                        rb[i, pl.ds(l, 16)] = jnp.where(ok, rel * FF + cc, 0)
                        vb[i, pl.ds(l, 16)] = jnp.where(
                            ok, v, jnp.zeros_like(v))

                for t in range(16):
                    pltpu.sync_copy(vb.at[t], acc.at[rb.at[t]], add=True)

            plsc.subcore_barrier()

            @pl.loop(0, 10)
            def _out(i):
                off = s * 100000 + i * 10000
                pltpu.sync_copy(acc.at[pl.ds(off, 10000)], stage2)
                pltpu.sync_copy(stage2,
                                x_hbm.at[pl.ds(q * QFLAT + off, 10000)])

    return k(frow, fcol, fval)


def _mlp(x, w1, b1, w2, b2):
    bm = 2000
    grid = (NN // bm,)

    def body(x_ref, w1_ref, b1_ref, w2_ref, b2_ref, z0_ref, z1_ref):
        h = lax.dot_general(
            x_ref[...], w1_ref[...], (((1,), (0,)), ((), ())),
            precision=lax.Precision.HIGHEST,
            preferred_element_type=jnp.float32)
        h = jnp.maximum(h + b1_ref[...], 0.0)
        z = lax.dot_general(
            h, w2_ref[...], (((1,), (0,)), ((), ())),
            precision=lax.Precision.HIGHEST,
            preferred_element_type=jnp.float32) + b2_ref[...]
        z0_ref[...] = z[:, :HALF].astype(jnp.bfloat16)
        z1_ref[...] = z[:, HALF:].astype(jnp.bfloat16)

    return pl.pallas_call(
        body,
        grid=grid,
        in_specs=[
            pl.BlockSpec((bm, FF), lambda i: (i, 0)),
            pl.BlockSpec((FF, HH), lambda i: (0, 0)),
            pl.BlockSpec((1, HH), lambda i: (0, 0)),
            pl.BlockSpec((HH, LL), lambda i: (0, 0)),
            pl.BlockSpec((1, LL), lambda i: (0, 0)),
        ],
        out_specs=[
            pl.BlockSpec((bm, HALF), lambda i: (i, 0)),
            pl.BlockSpec((bm, HALF), lambda i: (i, 0)),
        ],
        out_shape=[
            jax.ShapeDtypeStruct((NN, HALF), jnp.bfloat16),
            jax.ShapeDtypeStruct((NN, HALF), jnp.bfloat16),
        ],
    )(x, w1, b1, w2, b2)


def _propagate(z0, z1, src, dst, w):
    @functools.partial(
        pl.kernel,
        out_type=(
            jax.ShapeDtypeStruct((NN, HALF), jnp.bfloat16),
            jax.ShapeDtypeStruct((NN, HALF), jnp.bfloat16),
        ),
        mesh=_sc_mesh(),
        compiler_params=_sc_params(),
        scratch_types=[
            pltpu.VMEM((8, 128), jnp.int32),
            pltpu.VMEM((8, 128), jnp.int32),
            pltpu.VMEM((8, 128), jnp.float32),
            pltpu.VMEM((8, 128), jnp.int32),
            pltpu.VMEM((8, 128), jnp.int32),
            pltpu.VMEM((8, 128), jnp.float32),
            pltpu.VMEM((1024, HALF), jnp.bfloat16),
            pltpu.VMEM((1024, HALF), jnp.bfloat16),
            pltpu.VMEM((BR, HALF), jnp.bfloat16),
            pltpu.VMEM((BR, HALF), jnp.bfloat16),
            pltpu.VMEM((BR, HALF), jnp.bfloat16),
            pltpu.VMEM_SHARED((NN, HALF), jnp.bfloat16),
            pltpu.SemaphoreType.DMA,
            pltpu.SemaphoreType.DMA,
            pltpu.SemaphoreType.DMA,
            pltpu.SemaphoreType.DMA,
        ],
    )
    def k(z0_hbm, z1_hbm, src_hbm, dst_hbm, w_hbm, l0_hbm, l1_hbm,
          sb0, db0, wb0, sb1, db1, wb1, rows0, rows1, ba, bb, zb, acc,
          gs0, gs1, ss0, ss1):
        c = lax.axis_index("c")
        s = lax.axis_index("s")
        base = s * KTILE

        @pl.loop(0, BR)
        def _zzb(r):
            zb[r, :] = jnp.zeros((HALF,), jnp.bfloat16)

        def load_idx(cidx, sbx, dbx, wbx):
            roff = cidx * 8
            pltpu.sync_copy(src_hbm.at[pl.ds(roff, 8)], sbx)
            pltpu.sync_copy(dst_hbm.at[pl.ds(roff, 8)], dbx)
            pltpu.sync_copy(w_hbm.at[pl.ds(roff, 8)], wbx)

        def fire_gathers(l_ref, sbx, rowsx, gsx):
            for t in range(8):
                pltpu.async_copy(l_ref.at[sbx.at[t]],
                                 rowsx.at[pl.ds(t * 128, 128)], gsx)

        def fire_scatters(rowsx, dbx, ssx):
            for t in range(8):
                pltpu.async_copy(rowsx.at[pl.ds(t * 128, 128)],
                                 acc.at[dbx.at[t]], ssx, add=True)

        def drain(z_ref, bufx, semx):
            pltpu.make_async_copy(z_ref.at[pl.ds(0, 1024)], bufx, semx).wait()

        def scale(rowsx, wbx):
            @pl.loop(0, 1024, step=16)
            def _scale(e0):
                tv = jnp.broadcast_to(lax.shift_right_logical(e0, 7), (16,))
                l0 = lax.bitwise_and(e0, 127)
                for i in range(16):
                    ws = plsc.load_gather(
                        wbx, [tv, jnp.broadcast_to(l0 + i, (16,))])
                    wsb = plsc.pack(ws, ws,
                                    format=plsc.PackFormat.INTERLEAVED)
                    rowsx[e0 + i, :] = rowsx[e0 + i, :] * wsb

        def work(z_ref, l_ref):
            @pl.loop(s, NN // BR, step=NSUB)
            def _pro(kk):
                r0 = kk * BR
                pltpu.sync_copy(z_ref.at[pl.ds(r0, BR)], ba)
                pltpu.sync_copy(ba, l_ref.at[pl.ds(r0, BR)])
                pltpu.sync_copy(zb, acc.at[pl.ds(r0, BR)])

            plsc.subcore_barrier()

            @pl.loop(0, IT)
            def _iter(it):
                load_idx(base, sb0, db0, wb0)
                fire_gathers(l_ref, sb0, rows0, gs0)
                load_idx(base + 1, sb1, db1, wb1)
                fire_gathers(l_ref, sb1, rows1, gs1)

                @pl.loop(0, KTILE // 2)
                def _pair(g):
                    c0 = base + 2 * g
                    drain(z_ref, rows0, gs0)
                    scale(rows0, wb0)
                    fire_scatters(rows0, db0, ss0)
                    drain(z_ref, rows1, gs1)
                    scale(rows1, wb1)
                    fire_scatters(rows1, db1, ss1)

                    @pl.when(g < KTILE // 2 - 1)
                    def _prep():
                        drain(z_ref, rows0, ss0)
                        load_idx(c0 + 2, sb0, db0, wb0)
                        fire_gathers(l_ref, sb0, rows0, gs0)
                        drain(z_ref, rows1, ss1)
                        load_idx(c0 + 3, sb1, db1, wb1)
                        fire_gathers(l_ref, sb1, rows1, gs1)

                drain(z_ref, rows0, ss0)
                drain(z_ref, rows1, ss1)

                plsc.subcore_barrier()

                @pl.loop(s, NN // BR, step=NSUB)
                def _blend(kk):
                    r0 = kk * BR
                    pltpu.sync_copy(acc.at[pl.ds(r0, BR)], ba)
                    pltpu.sync_copy(z_ref.at[pl.ds(r0, BR)], bb)

                    @pl.loop(0, BR)
                    def _rowb(r):
                        ba[r, :] = (1.0 - AL) * ba[r, :] + AL * bb[r, :]

                    pltpu.sync_copy(ba, l_ref.at[pl.ds(r0, BR)])
                    pltpu.sync_copy(zb, acc.at[pl.ds(r0, BR)])

                plsc.subcore_barrier()

        @pl.when(c == 0)
        def _():
            work(z0_hbm, l0_hbm)

        @pl.when(c == 1)
        def _():
            work(z1_hbm, l1_hbm)

    return k(z0, z1, src, dst, w)


def _log_softmax(l0, l1):
    bm = 2000
    grid = (NN // bm,)

    def body(l0_ref, l1_ref, o_ref):
        x = jnp.concatenate(
            [l0_ref[...], l1_ref[...]], axis=1).astype(jnp.float32)
        m = jnp.max(x, axis=1, keepdims=True)
        xm = x - m
        lse = jnp.log(jnp.sum(jnp.exp(xm), axis=1, keepdims=True))
        o_ref[...] = xm - lse

    return pl.pallas_call(
        body,
        grid=grid,
        in_specs=[
            pl.BlockSpec((bm, HALF), lambda i: (i, 0)),
            pl.BlockSpec((bm, HALF), lambda i: (i, 0)),
        ],
        out_specs=pl.BlockSpec((bm, LL), lambda i: (i, 0)),
        out_shape=jax.ShapeDtypeStruct((NN, LL), jnp.float32),
    )(l0, l1)


def kernel(feature_indices, feature_values, edge_indices, edge_weights,
           W1, b1, W2, b2):
    frow = feature_indices[0].astype(jnp.int32)
    fcol = feature_indices[1].astype(jnp.int32)
    fval = feature_values.astype(jnp.float32)
    fpad = F_ROWS * 128 - ZF
    frow = jnp.concatenate([frow, jnp.zeros((fpad,), jnp.int32)]).reshape(
        F_ROWS, 128)
    fcol = jnp.concatenate([fcol, jnp.zeros((fpad,), jnp.int32)]).reshape(
        F_ROWS, 128)
    fval = jnp.concatenate([fval, jnp.zeros((fpad,), jnp.float32)]).reshape(
        F_ROWS, 128)

    src = edge_indices[1].astype(jnp.int32)
    dst = edge_indices[0].astype(jnp.int32)
    wgt = edge_weights.astype(jnp.float32)
    epad = E_ROWS * 128 - EE
    src = jnp.concatenate([src, jnp.zeros((epad,), jnp.int32)]).reshape(
        E_ROWS, 128)
    dst = jnp.concatenate([dst, jnp.zeros((epad,), jnp.int32)]).reshape(
        E_ROWS, 128)
    wgt = jnp.concatenate([wgt, jnp.zeros((epad,), jnp.float32)]).reshape(
        E_ROWS, 128)

    x = _densify(frow, fcol, fval).reshape(NN, FF)
    z0, z1 = _mlp(x, W1, b1.reshape(1, HH), W2, b2.reshape(1, LL))
    l0, l1 = _propagate(z0, z1, src, dst, wgt)
    return _log_softmax(l0, l1)


# speedup vs baseline: 7.8242x; 1.2901x over previous
import dataclasses
import functools

import jax
import jax.numpy as jnp
from jax import lax
from jax.experimental import pallas as pl
from jax.experimental.pallas import tpu as pltpu
from jax.experimental.pallas import tpu_sc as plsc

NN = 50000
FF = 128
HH = 128
LL = 64
EE = 800000
ZF = 1000000
AL = 0.1
IT = 10

NSUB = 16
HALF = LL // 2
QROWS = NN // 4
QFLAT = QROWS * FF

F_ROWS = 7824
E_ROWS = 6400
F_CHUNKS = F_ROWS // 16
E_CHUNKS = E_ROWS // 8
KTILE = E_CHUNKS // NSUB

BR = 80


def _sc_mesh():
    return plsc.VectorSubcoreMesh(core_axis_name="c", subcore_axis_name="s")


def _sc_params():
    cp = pltpu.CompilerParams()
    if "needs_layout_passes" in pltpu.CompilerParams.__dataclass_fields__:
        cp = dataclasses.replace(cp, needs_layout_passes=False)
    if "use_tc_tiling_on_sc" in pltpu.CompilerParams.__dataclass_fields__:
        cp = dataclasses.replace(cp, use_tc_tiling_on_sc=False)
    return cp


def _densify(frow, fcol, fval):
    @functools.partial(
        pl.kernel,
        out_type=jax.ShapeDtypeStruct((NN * FF,), jnp.float32),
        mesh=_sc_mesh(),
        scratch_types=[
            pltpu.VMEM((16, 128), jnp.int32),
            pltpu.VMEM((16, 128), jnp.int32),
            pltpu.VMEM((16, 128), jnp.float32),
            pltpu.VMEM((10000,), jnp.float32),
            pltpu.VMEM((10000,), jnp.float32),
            pltpu.VMEM_SHARED((QFLAT,), jnp.float32),
            pltpu.SemaphoreType.DMA,
        ],
    )
    def k(frow_hbm, fcol_hbm, fval_hbm, x_hbm, rb, cb, vb, stage, stage2,
          acc, sem):
        c = lax.axis_index("c")
        s = lax.axis_index("s")

        @pl.loop(0, 10000, step=16)
        def _zstage(i):
            stage[pl.ds(i, 16)] = jnp.zeros((16,), jnp.float32)

        @pl.loop(0, 2)
        def _pass(p):
            q = 2 * c + p
            base = q * QROWS

            @pl.loop(0, 10)
            def _z(i):
                pltpu.sync_copy(
                    stage, acc.at[pl.ds(s * 100000 + i * 10000, 10000)])

            plsc.subcore_barrier()

            @pl.loop(s, F_CHUNKS, step=NSUB)
            def _chunk(j):
                roff = j * 16
                pltpu.sync_copy(frow_hbm.at[pl.ds(roff, 16)], rb)
                pltpu.sync_copy(fcol_hbm.at[pl.ds(roff, 16)], cb)
                pltpu.sync_copy(fval_hbm.at[pl.ds(roff, 16)], vb)

                @pl.loop(0, 16)
                def _row(i):
                    @pl.loop(0, 128, step=16)
                    def _lane(l):
                        r = rb[i, pl.ds(l, 16)]
                        cc = cb[i, pl.ds(l, 16)]
                        v = vb[i, pl.ds(l, 16)]
                        rel = r - base
                        ok = (rel >= 0) & (rel < QROWS)
                        rb[i, pl.ds(l, 16)] = jnp.where(ok, rel * FF + cc, 0)
                        vb[i, pl.ds(l, 16)] = jnp.where(
                            ok, v, jnp.zeros_like(v))

                for t in range(16):
                    pltpu.sync_copy(vb.at[t], acc.at[rb.at[t]], add=True)

            plsc.subcore_barrier()

            @pl.loop(0, 10)
            def _out(i):
                off = s * 100000 + i * 10000
                pltpu.sync_copy(acc.at[pl.ds(off, 10000)], stage2)
                pltpu.sync_copy(stage2,
                                x_hbm.at[pl.ds(q * QFLAT + off, 10000)])

    return k(frow, fcol, fval)


def _mlp(x, w1, b1, w2, b2):
    bm = 2000
    grid = (NN // bm,)

    def body(x_ref, w1_ref, b1_ref, w2_ref, b2_ref, z0_ref, z1_ref):
        h = lax.dot_general(
            x_ref[...], w1_ref[...], (((1,), (0,)), ((), ())),
            precision=lax.Precision.HIGHEST,
            preferred_element_type=jnp.float32)
        h = jnp.maximum(h + b1_ref[...], 0.0)
        z = lax.dot_general(
            h, w2_ref[...], (((1,), (0,)), ((), ())),
            precision=lax.Precision.HIGHEST,
            preferred_element_type=jnp.float32) + b2_ref[...]
        z0_ref[...] = z[:, :HALF].astype(jnp.bfloat16)
        z1_ref[...] = z[:, HALF:].astype(jnp.bfloat16)

    return pl.pallas_call(
        body,
        grid=grid,
        in_specs=[
            pl.BlockSpec((bm, FF), lambda i: (i, 0)),
            pl.BlockSpec((FF, HH), lambda i: (0, 0)),
            pl.BlockSpec((1, HH), lambda i: (0, 0)),
            pl.BlockSpec((HH, LL), lambda i: (0, 0)),
            pl.BlockSpec((1, LL), lambda i: (0, 0)),
        ],
        out_specs=[
            pl.BlockSpec((bm, HALF), lambda i: (i, 0)),
            pl.BlockSpec((bm, HALF), lambda i: (i, 0)),
        ],
        out_shape=[
            jax.ShapeDtypeStruct((NN, HALF), jnp.bfloat16),
            jax.ShapeDtypeStruct((NN, HALF), jnp.bfloat16),
        ],
    )(x, w1, b1, w2, b2)


def _propagate(z0, z1, src, dst, w):
    @functools.partial(
        pl.kernel,
        out_type=(
            jax.ShapeDtypeStruct((NN, HALF), jnp.bfloat16),
            jax.ShapeDtypeStruct((NN, HALF), jnp.bfloat16),
        ),
        mesh=_sc_mesh(),
        compiler_params=_sc_params(),
        scratch_types=[
            pltpu.VMEM((8, 128), jnp.int32),
            pltpu.VMEM((8, 128), jnp.int32),
            pltpu.VMEM((1024, HALF), jnp.bfloat16),
            pltpu.VMEM((8, 128), jnp.int32),
            pltpu.VMEM((8, 128), jnp.int32),
            pltpu.VMEM((1024, HALF), jnp.bfloat16),
            pltpu.VMEM((1024, HALF), jnp.bfloat16),
            pltpu.VMEM((1024, HALF), jnp.bfloat16),
            pltpu.VMEM((BR, HALF), jnp.bfloat16),
            pltpu.VMEM((BR, HALF), jnp.bfloat16),
            pltpu.VMEM((BR, HALF), jnp.bfloat16),
            pltpu.VMEM_SHARED((NN, HALF), jnp.bfloat16),
            pltpu.SemaphoreType.DMA,
            pltpu.SemaphoreType.DMA,
            pltpu.SemaphoreType.DMA,
            pltpu.SemaphoreType.DMA,
            pltpu.SemaphoreType.DMA,
            pltpu.SemaphoreType.DMA,
        ],
    )
    def k(z0_hbm, z1_hbm, src_hbm, dst_hbm, w_hbm, l0_hbm, l1_hbm,
          sb0, db0, wb0, sb1, db1, wb1, rows0, rows1, ba, bb, zb, acc,
          gs0, gs1, ss0, ss1, ws0, ws1):
        c = lax.axis_index("c")
        s = lax.axis_index("s")
        base = s * KTILE

        @pl.loop(0, BR)
        def _zzb(r):
            zb[r, :] = jnp.zeros((HALF,), jnp.bfloat16)

        def load_idx(cidx, sbx, dbx, wbx, wsx):
            roff = cidx * 8
            pltpu.async_copy(w_hbm.at[pl.ds(cidx * 1024, 1024)], wbx, wsx)
            pltpu.sync_copy(src_hbm.at[pl.ds(roff, 8)], sbx)
            pltpu.sync_copy(dst_hbm.at[pl.ds(roff, 8)], dbx)

        def wait_w(wbx, wsx):
            pltpu.make_async_copy(
                w_hbm.at[pl.ds(0, 1024)], wbx, wsx).wait()

        def fire_gathers(l_ref, sbx, rowsx, gsx):
            for t in range(8):
                pltpu.async_copy(l_ref.at[sbx.at[t]],
                                 rowsx.at[pl.ds(t * 128, 128)], gsx)

        def fire_scatters(rowsx, dbx, ssx):
            for t in range(8):
                pltpu.async_copy(rowsx.at[pl.ds(t * 128, 128)],
                                 acc.at[dbx.at[t]], ssx, add=True)

        def drain(z_ref, bufx, semx):
            pltpu.make_async_copy(z_ref.at[pl.ds(0, 1024)], bufx, semx).wait()

        def scale(rowsx, wbx):
            @pl.loop(0, 1024, step=8)
            def _scale(e0):
                for i in range(8):
                    rowsx[e0 + i, :] = rowsx[e0 + i, :] * wbx[e0 + i, :]

        def work(z_ref, l_ref):
            @pl.loop(s, NN // BR, step=NSUB)
            def _pro(kk):
                r0 = kk * BR
                pltpu.sync_copy(z_ref.at[pl.ds(r0, BR)], ba)
                pltpu.sync_copy(ba, l_ref.at[pl.ds(r0, BR)])
                pltpu.sync_copy(zb, acc.at[pl.ds(r0, BR)])

            plsc.subcore_barrier()

            @pl.loop(0, IT)
            def _iter(it):
                load_idx(base, sb0, db0, wb0, ws0)
                fire_gathers(l_ref, sb0, rows0, gs0)
                load_idx(base + 1, sb1, db1, wb1, ws1)
                fire_gathers(l_ref, sb1, rows1, gs1)

                @pl.loop(0, KTILE // 2)
                def _pair(g):
                    c0 = base + 2 * g
                    drain(z_ref, rows0, gs0)
                    wait_w(wb0, ws0)
                    scale(rows0, wb0)
                    fire_scatters(rows0, db0, ss0)
                    drain(z_ref, rows1, gs1)
                    wait_w(wb1, ws1)
                    scale(rows1, wb1)
                    fire_scatters(rows1, db1, ss1)

                    @pl.when(g < KTILE // 2 - 1)
                    def _prep():
                        drain(z_ref, rows0, ss0)
                        load_idx(c0 + 2, sb0, db0, wb0, ws0)
                        fire_gathers(l_ref, sb0, rows0, gs0)
                        drain(z_ref, rows1, ss1)
                        load_idx(c0 + 3, sb1, db1, wb1, ws1)
                        fire_gathers(l_ref, sb1, rows1, gs1)

                drain(z_ref, rows0, ss0)
                drain(z_ref, rows1, ss1)

                plsc.subcore_barrier()

                @pl.loop(s, NN // BR, step=NSUB)
                def _blend(kk):
                    r0 = kk * BR
                    pltpu.sync_copy(acc.at[pl.ds(r0, BR)], ba)
                    pltpu.sync_copy(z_ref.at[pl.ds(r0, BR)], bb)

                    @pl.loop(0, BR)
                    def _rowb(r):
                        ba[r, :] = (1.0 - AL) * ba[r, :] + AL * bb[r, :]

                    pltpu.sync_copy(ba, l_ref.at[pl.ds(r0, BR)])
                    pltpu.sync_copy(zb, acc.at[pl.ds(r0, BR)])

                plsc.subcore_barrier()

        @pl.when(c == 0)
        def _():
            work(z0_hbm, l0_hbm)

        @pl.when(c == 1)
        def _():
            work(z1_hbm, l1_hbm)

    return k(z0, z1, src, dst, w)


def _log_softmax(l0, l1):
    bm = 2000
    grid = (NN // bm,)

    def body(l0_ref, l1_ref, o_ref):
        x = jnp.concatenate(
            [l0_ref[...], l1_ref[...]], axis=1).astype(jnp.float32)
        m = jnp.max(x, axis=1, keepdims=True)
        xm = x - m
        lse = jnp.log(jnp.sum(jnp.exp(xm), axis=1, keepdims=True))
        o_ref[...] = xm - lse

    return pl.pallas_call(
        body,
        grid=grid,
        in_specs=[
            pl.BlockSpec((bm, HALF), lambda i: (i, 0)),
            pl.BlockSpec((bm, HALF), lambda i: (i, 0)),
        ],
        out_specs=pl.BlockSpec((bm, LL), lambda i: (i, 0)),
        out_shape=jax.ShapeDtypeStruct((NN, LL), jnp.float32),
    )(l0, l1)


def kernel(feature_indices, feature_values, edge_indices, edge_weights,
           W1, b1, W2, b2):
    frow = feature_indices[0].astype(jnp.int32)
    fcol = feature_indices[1].astype(jnp.int32)
    fval = feature_values.astype(jnp.float32)
    fpad = F_ROWS * 128 - ZF
    frow = jnp.concatenate([frow, jnp.zeros((fpad,), jnp.int32)]).reshape(
        F_ROWS, 128)
    fcol = jnp.concatenate([fcol, jnp.zeros((fpad,), jnp.int32)]).reshape(
        F_ROWS, 128)
    fval = jnp.concatenate([fval, jnp.zeros((fpad,), jnp.float32)]).reshape(
        F_ROWS, 128)

    src = edge_indices[1].astype(jnp.int32)
    dst = edge_indices[0].astype(jnp.int32)
    wgt = edge_weights.astype(jnp.float32)
    epad = E_ROWS * 128 - EE
    src = jnp.concatenate([src, jnp.zeros((epad,), jnp.int32)]).reshape(
        E_ROWS, 128)
    dst = jnp.concatenate([dst, jnp.zeros((epad,), jnp.int32)]).reshape(
        E_ROWS, 128)
    wgt = jnp.concatenate([wgt, jnp.zeros((epad,), jnp.float32)])
    wgt = jnp.broadcast_to(
        wgt.astype(jnp.bfloat16)[:, None], (E_ROWS * 128, HALF))

    x = _densify(frow, fcol, fval).reshape(NN, FF)
    z0, z1 = _mlp(x, W1, b1.reshape(1, HH), W2, b2.reshape(1, LL))
    l0, l1 = _propagate(z0, z1, src, dst, wgt)
    return _log_softmax(l0, l1)


# speedup vs baseline: 9.5947x; 1.2263x over previous
import dataclasses
import functools

import jax
import jax.numpy as jnp
from jax import lax
from jax.experimental import pallas as pl
from jax.experimental.pallas import tpu as pltpu
from jax.experimental.pallas import tpu_sc as plsc

NN = 50000
FF = 128
HH = 128
LL = 64
EE = 800000
ZF = 1000000
AL = 0.1
IT = 10

NSUB = 16
HALF = LL // 2
QROWS = NN // 4
QFLAT = QROWS * FF

F_ROWS = 7824
E_ROWS = 6400
F_CHUNKS = F_ROWS // 16
E_CHUNKS = E_ROWS // 8
KTILE = E_CHUNKS // NSUB

BR = 80


def _sc_mesh():
    return plsc.VectorSubcoreMesh(core_axis_name="c", subcore_axis_name="s")


def _sc_params():
    cp = pltpu.CompilerParams()
    if "needs_layout_passes" in pltpu.CompilerParams.__dataclass_fields__:
        cp = dataclasses.replace(cp, needs_layout_passes=False)
    if "use_tc_tiling_on_sc" in pltpu.CompilerParams.__dataclass_fields__:
        cp = dataclasses.replace(cp, use_tc_tiling_on_sc=False)
    return cp


def _densify(frow, fcol, fval):
    @functools.partial(
        pl.kernel,
        out_type=jax.ShapeDtypeStruct((NN * FF,), jnp.float32),
        mesh=_sc_mesh(),
        scratch_types=[
            pltpu.VMEM((16, 128), jnp.int32),
            pltpu.VMEM((16, 128), jnp.int32),
            pltpu.VMEM((16, 128), jnp.float32),
            pltpu.VMEM((10000,), jnp.float32),
            pltpu.VMEM((10000,), jnp.float32),
            pltpu.VMEM_SHARED((QFLAT + 128,), jnp.float32),
            pltpu.SemaphoreType.DMA,
        ],
    )
    def k(frow_hbm, fcol_hbm, fval_hbm, x_hbm, rb, cb, vb, stage, stage2,
          acc, sem):
        c = lax.axis_index("c")
        s = lax.axis_index("s")

        @pl.loop(0, 10000, step=16)
        def _zstage(i):
            stage[pl.ds(i, 16)] = jnp.zeros((16,), jnp.float32)

        @pl.loop(0, 2)
        def _pass(p):
            q = 2 * c + p
            base = q * QROWS

            @pl.loop(0, 10)
            def _z(i):
                pltpu.sync_copy(
                    stage, acc.at[pl.ds(s * 100000 + i * 10000, 10000)])

            plsc.subcore_barrier()

            @pl.loop(s, F_CHUNKS, step=NSUB)
            def _chunk(j):
                roff = j * 16
                pltpu.sync_copy(frow_hbm.at[pl.ds(roff, 16)], rb)
                pltpu.sync_copy(fcol_hbm.at[pl.ds(roff, 16)], cb)
                pltpu.sync_copy(fval_hbm.at[pl.ds(roff, 16)], vb)

                @pl.loop(0, 16)
                def _row(i):
                    @pl.loop(0, 128, step=16)
                    def _lane(l):
                        r = rb[i, pl.ds(l, 16)]
                        cc = cb[i, pl.ds(l, 16)]
                        rel = r - base
                        ok = (rel >= 0) & (rel < QROWS)
                        rb[i, pl.ds(l, 16)] = jnp.where(
                            ok, rel * FF + cc, QFLAT + cc)

                for t in range(16):
                    pltpu.sync_copy(vb.at[t], acc.at[rb.at[t]], add=True)

            plsc.subcore_barrier()

            @pl.loop(0, 10)
            def _out(i):
                off = s * 100000 + i * 10000
                pltpu.sync_copy(acc.at[pl.ds(off, 10000)], stage2)
                pltpu.sync_copy(stage2,
                                x_hbm.at[pl.ds(q * QFLAT + off, 10000)])

    return k(frow, fcol, fval)


def _mlp(x, w1, b1, w2, b2):
    bm = 2000
    grid = (NN // bm,)

    def body(x_ref, w1_ref, b1_ref, w2_ref, b2_ref, z0_ref, z1_ref):
        h = lax.dot_general(
            x_ref[...].astype(jnp.float32), w1_ref[...],
            (((1,), (0,)), ((), ())),
            precision=lax.Precision.HIGHEST,
            preferred_element_type=jnp.float32)
        h = jnp.maximum(h + b1_ref[...], 0.0)
        z = lax.dot_general(
            h, w2_ref[...], (((1,), (0,)), ((), ())),
            precision=lax.Precision.HIGHEST,
            preferred_element_type=jnp.float32) + b2_ref[...]
        z0_ref[...] = z[:, :HALF].astype(jnp.bfloat16)
        z1_ref[...] = z[:, HALF:].astype(jnp.bfloat16)

    return pl.pallas_call(
        body,
        grid=grid,
        in_specs=[
            pl.BlockSpec((bm, FF), lambda i: (i, 0)),
            pl.BlockSpec((FF, HH), lambda i: (0, 0)),
            pl.BlockSpec((1, HH), lambda i: (0, 0)),
            pl.BlockSpec((HH, LL), lambda i: (0, 0)),
            pl.BlockSpec((1, LL), lambda i: (0, 0)),
        ],
        out_specs=[
            pl.BlockSpec((bm, HALF), lambda i: (i, 0)),
            pl.BlockSpec((bm, HALF), lambda i: (i, 0)),
        ],
        out_shape=[
            jax.ShapeDtypeStruct((NN, HALF), jnp.bfloat16),
            jax.ShapeDtypeStruct((NN, HALF), jnp.bfloat16),
        ],
    )(x, w1, b1, w2, b2)


def _propagate(z0, z1, src, dst, w):
    @functools.partial(
        pl.kernel,
        out_type=(
            jax.ShapeDtypeStruct((NN, HALF), jnp.bfloat16),
            jax.ShapeDtypeStruct((NN, HALF), jnp.bfloat16),
        ),
        mesh=_sc_mesh(),
        compiler_params=_sc_params(),
        scratch_types=[
            pltpu.VMEM((8, 128), jnp.int32),
            pltpu.VMEM((8, 128), jnp.int32),
            pltpu.VMEM((1024, HALF), jnp.bfloat16),
            pltpu.VMEM((8, 128), jnp.int32),
            pltpu.VMEM((8, 128), jnp.int32),
            pltpu.VMEM((1024, HALF), jnp.bfloat16),
            pltpu.VMEM((1024, HALF), jnp.bfloat16),
            pltpu.VMEM((1024, HALF), jnp.bfloat16),
            pltpu.VMEM((BR, HALF), jnp.bfloat16),
            pltpu.VMEM((BR, HALF), jnp.bfloat16),
            pltpu.VMEM((BR, HALF), jnp.bfloat16),
            pltpu.VMEM_SHARED((NN, HALF), jnp.bfloat16),
            pltpu.SemaphoreType.DMA,
            pltpu.SemaphoreType.DMA,
            pltpu.SemaphoreType.DMA,
            pltpu.SemaphoreType.DMA,
            pltpu.SemaphoreType.DMA,
            pltpu.SemaphoreType.DMA,
        ],
    )
    def k(z0_hbm, z1_hbm, src_hbm, dst_hbm, w_hbm, l0_hbm, l1_hbm,
          sb0, db0, wb0, sb1, db1, wb1, rows0, rows1, ba, bb, zb, acc,
          gs0, gs1, ss0, ss1, ws0, ws1):
        c = lax.axis_index("c")
        s = lax.axis_index("s")
        base = s * KTILE

        @pl.loop(0, BR)
        def _zzb(r):
            zb[r, :] = jnp.zeros((HALF,), jnp.bfloat16)

        def load_idx(cidx, sbx, dbx, wbx, wsx):
            roff = cidx * 8
            pltpu.async_copy(w_hbm.at[pl.ds(cidx * 1024, 1024)], wbx, wsx)
            pltpu.sync_copy(src_hbm.at[pl.ds(roff, 8)], sbx)
            pltpu.sync_copy(dst_hbm.at[pl.ds(roff, 8)], dbx)

        def wait_w(wbx, wsx):
            pltpu.make_async_copy(
                w_hbm.at[pl.ds(0, 1024)], wbx, wsx).wait()

        def fire_gathers(l_ref, sbx, rowsx, gsx):
            for t in range(8):
                pltpu.async_copy(l_ref.at[sbx.at[t]],
                                 rowsx.at[pl.ds(t * 128, 128)], gsx)

        def fire_scatters(rowsx, dbx, ssx):
            for t in range(8):
                pltpu.async_copy(rowsx.at[pl.ds(t * 128, 128)],
                                 acc.at[dbx.at[t]], ssx, add=True)

        def drain(z_ref, bufx, semx):
            pltpu.make_async_copy(z_ref.at[pl.ds(0, 1024)], bufx, semx).wait()

        def scale(rowsx, wbx):
            @pl.loop(0, 1024, step=8)
            def _scale(e0):
                for i in range(8):
                    rowsx[e0 + i, :] = rowsx[e0 + i, :] * wbx[e0 + i, :]

        def work(z_ref, l_ref):
            @pl.loop(s, NN // BR, step=NSUB)
            def _pro(kk):
                r0 = kk * BR
                pltpu.sync_copy(z_ref.at[pl.ds(r0, BR)], ba)
                pltpu.sync_copy(ba, l_ref.at[pl.ds(r0, BR)])
                pltpu.sync_copy(zb, acc.at[pl.ds(r0, BR)])

            plsc.subcore_barrier()

            @pl.loop(0, IT)
            def _iter(it):
                load_idx(base, sb0, db0, wb0, ws0)
                fire_gathers(l_ref, sb0, rows0, gs0)
                load_idx(base + 1, sb1, db1, wb1, ws1)
                fire_gathers(l_ref, sb1, rows1, gs1)

                @pl.loop(0, KTILE // 2)
                def _pair(g):
                    c0 = base + 2 * g
                    drain(z_ref, rows0, gs0)
                    wait_w(wb0, ws0)
                    scale(rows0, wb0)
                    fire_scatters(rows0, db0, ss0)
                    drain(z_ref, rows1, gs1)
                    wait_w(wb1, ws1)
                    scale(rows1, wb1)
                    fire_scatters(rows1, db1, ss1)

                    @pl.when(g < KTILE // 2 - 1)
                    def _prep():
                        drain(z_ref, rows0, ss0)
                        load_idx(c0 + 2, sb0, db0, wb0, ws0)
                        fire_gathers(l_ref, sb0, rows0, gs0)
                        drain(z_ref, rows1, ss1)
                        load_idx(c0 + 3, sb1, db1, wb1, ws1)
                        fire_gathers(l_ref, sb1, rows1, gs1)

                drain(z_ref, rows0, ss0)
                drain(z_ref, rows1, ss1)

                plsc.subcore_barrier()

                @pl.loop(s, NN // BR, step=NSUB)
                def _blend(kk):
                    r0 = kk * BR
                    pltpu.sync_copy(acc.at[pl.ds(r0, BR)], ba)
                    pltpu.sync_copy(z_ref.at[pl.ds(r0, BR)], bb)

                    @pl.loop(0, BR)
                    def _rowb(r):
                        ba[r, :] = (1.0 - AL) * ba[r, :] + AL * bb[r, :]

                    pltpu.sync_copy(ba, l_ref.at[pl.ds(r0, BR)])
                    pltpu.sync_copy(zb, acc.at[pl.ds(r0, BR)])

                plsc.subcore_barrier()

        @pl.when(c == 0)
        def _():
            work(z0_hbm, l0_hbm)

        @pl.when(c == 1)
        def _():
            work(z1_hbm, l1_hbm)

    return k(z0, z1, src, dst, w)


def _log_softmax(l0, l1):
    bm = 2000
    grid = (NN // bm,)

    def body(l0_ref, l1_ref, o_ref):
        x = jnp.concatenate(
            [l0_ref[...], l1_ref[...]], axis=1).astype(jnp.float32)
        m = jnp.max(x, axis=1, keepdims=True)
        xm = x - m
        lse = jnp.log(jnp.sum(jnp.exp(xm), axis=1, keepdims=True))
        o_ref[...] = xm - lse

    return pl.pallas_call(
        body,
        grid=grid,
        in_specs=[
            pl.BlockSpec((bm, HALF), lambda i: (i, 0)),
            pl.BlockSpec((bm, HALF), lambda i: (i, 0)),
        ],
        out_specs=pl.BlockSpec((bm, LL), lambda i: (i, 0)),
        out_shape=jax.ShapeDtypeStruct((NN, LL), jnp.float32),
    )(l0, l1)


def kernel(feature_indices, feature_values, edge_indices, edge_weights,
           W1, b1, W2, b2):
    frow = feature_indices[0].astype(jnp.int32)
    fcol = feature_indices[1].astype(jnp.int32)
    fval = feature_values.astype(jnp.float32)
    fpad = F_ROWS * 128 - ZF
    frow = jnp.concatenate([frow, jnp.zeros((fpad,), jnp.int32)]).reshape(
        F_ROWS, 128)
    fcol = jnp.concatenate([fcol, jnp.zeros((fpad,), jnp.int32)]).reshape(
        F_ROWS, 128)
    fval = jnp.concatenate([fval, jnp.zeros((fpad,), jnp.float32)]).reshape(
        F_ROWS, 128)

    src = edge_indices[1].astype(jnp.int32)
    dst = edge_indices[0].astype(jnp.int32)
    wgt = edge_weights.astype(jnp.float32)
    epad = E_ROWS * 128 - EE
    src = jnp.concatenate([src, jnp.zeros((epad,), jnp.int32)]).reshape(
        E_ROWS, 128)
    dst = jnp.concatenate([dst, jnp.zeros((epad,), jnp.int32)]).reshape(
        E_ROWS, 128)
    wgt = jnp.concatenate([wgt, jnp.zeros((epad,), jnp.float32)])
    wgt = jnp.broadcast_to(
        wgt.astype(jnp.bfloat16)[:, None], (E_ROWS * 128, HALF))

    x = _densify(frow, fcol, fval).reshape(NN, FF)
    z0, z1 = _mlp(x, W1, b1.reshape(1, HH), W2, b2.reshape(1, LL))
    l0, l1 = _propagate(z0, z1, src, dst, wgt)
    return _log_softmax(l0, l1)


# speedup vs baseline: 9.6083x; 1.0014x over previous
import dataclasses
import functools

import jax
import jax.numpy as jnp
from jax import lax
from jax.experimental import pallas as pl
from jax.experimental.pallas import tpu as pltpu
from jax.experimental.pallas import tpu_sc as plsc

NN = 50000
FF = 128
HH = 128
LL = 64
EE = 800000
ZF = 1000000
AL = 0.1
IT = 10

NSUB = 16
HALF = LL // 2
QROWS = NN // 4
QFLAT = QROWS * FF

F_ROWS = 7824
E_ROWS = 6400
F_CHUNKS = F_ROWS // 16
E_CHUNKS = E_ROWS // 8
KTILE = E_CHUNKS // NSUB

BR = 80


def _sc_mesh():
    return plsc.VectorSubcoreMesh(core_axis_name="c", subcore_axis_name="s")


def _sc_params():
    cp = pltpu.CompilerParams()
    if "needs_layout_passes" in pltpu.CompilerParams.__dataclass_fields__:
        cp = dataclasses.replace(cp, needs_layout_passes=False)
    if "use_tc_tiling_on_sc" in pltpu.CompilerParams.__dataclass_fields__:
        cp = dataclasses.replace(cp, use_tc_tiling_on_sc=False)
    return cp


def _densify(frow, fcol, fval):
    @functools.partial(
        pl.kernel,
        out_type=jax.ShapeDtypeStruct((NN * FF,), jnp.float32),
        mesh=_sc_mesh(),
        scratch_types=[
            pltpu.VMEM((16, 128), jnp.int32),
            pltpu.VMEM((16, 128), jnp.int32),
            pltpu.VMEM((16, 128), jnp.float32),
            pltpu.VMEM((10000,), jnp.float32),
            pltpu.VMEM((10000,), jnp.float32),
            pltpu.VMEM_SHARED((QFLAT + 128,), jnp.float32),
            pltpu.SemaphoreType.DMA,
        ],
    )
    def k(frow_hbm, fcol_hbm, fval_hbm, x_hbm, rb, cb, vb, stage, stage2,
          acc, sem):
        c = lax.axis_index("c")
        s = lax.axis_index("s")

        @pl.loop(0, 10000, step=16)
        def _zstage(i):
            stage[pl.ds(i, 16)] = jnp.zeros((16,), jnp.float32)

        @pl.loop(0, 2)
        def _pass(p):
            q = 2 * c + p
            base = q * QROWS

            @pl.loop(0, 10)
            def _z(i):
                pltpu.sync_copy(
                    stage, acc.at[pl.ds(s * 100000 + i * 10000, 10000)])

            plsc.subcore_barrier()

            @pl.loop(s, F_CHUNKS, step=NSUB)
            def _chunk(j):
                roff = j * 16
                pltpu.sync_copy(frow_hbm.at[pl.ds(roff, 16)], rb)
                pltpu.sync_copy(fcol_hbm.at[pl.ds(roff, 16)], cb)
                pltpu.sync_copy(fval_hbm.at[pl.ds(roff, 16)], vb)

                @pl.loop(0, 16)
                def _row(i):
                    @pl.loop(0, 128, step=16)
                    def _lane(l):
                        r = rb[i, pl.ds(l, 16)]
                        cc = cb[i, pl.ds(l, 16)]
                        rel = r - base
                        ok = (rel >= 0) & (rel < QROWS)
                        rb[i, pl.ds(l, 16)] = jnp.where(
                            ok, rel * FF + cc, QFLAT + cc)

                for t in range(16):
                    pltpu.sync_copy(vb.at[t], acc.at[rb.at[t]], add=True)

            plsc.subcore_barrier()

            @pl.loop(0, 10)
            def _out(i):
                off = s * 100000 + i * 10000
                pltpu.sync_copy(acc.at[pl.ds(off, 10000)], stage2)
                pltpu.sync_copy(stage2,
                                x_hbm.at[pl.ds(q * QFLAT + off, 10000)])

    return k(frow, fcol, fval)


def _mlp(x, w1, b1, w2, b2):
    bm = 2000
    grid = (NN // bm,)

    def body(x_ref, w1_ref, b1_ref, w2_ref, b2_ref, z0_ref, z1_ref):
        h = lax.dot_general(
            x_ref[...].astype(jnp.float32), w1_ref[...],
            (((1,), (0,)), ((), ())),
            precision=lax.Precision.HIGHEST,
            preferred_element_type=jnp.float32)
        h = jnp.maximum(h + b1_ref[...], 0.0)
        z = lax.dot_general(
            h, w2_ref[...], (((1,), (0,)), ((), ())),
            precision=lax.Precision.HIGHEST,
            preferred_element_type=jnp.float32) + b2_ref[...]
        z0_ref[...] = z[:, :HALF].astype(jnp.bfloat16)
        z1_ref[...] = z[:, HALF:].astype(jnp.bfloat16)

    return pl.pallas_call(
        body,
        grid=grid,
        in_specs=[
            pl.BlockSpec((bm, FF), lambda i: (i, 0)),
            pl.BlockSpec((FF, HH), lambda i: (0, 0)),
            pl.BlockSpec((1, HH), lambda i: (0, 0)),
            pl.BlockSpec((HH, LL), lambda i: (0, 0)),
            pl.BlockSpec((1, LL), lambda i: (0, 0)),
        ],
        out_specs=[
            pl.BlockSpec((bm, HALF), lambda i: (i, 0)),
            pl.BlockSpec((bm, HALF), lambda i: (i, 0)),
        ],
        out_shape=[
            jax.ShapeDtypeStruct((NN, HALF), jnp.bfloat16),
            jax.ShapeDtypeStruct((NN, HALF), jnp.bfloat16),
        ],
    )(x, w1, b1, w2, b2)


def _propagate(z0, z1, src, dst, w):
    @functools.partial(
        pl.kernel,
        out_type=(
            jax.ShapeDtypeStruct((NN, HALF), jnp.bfloat16),
            jax.ShapeDtypeStruct((NN, HALF), jnp.bfloat16),
        ),
        mesh=_sc_mesh(),
        compiler_params=_sc_params(),
        scratch_types=[
            pltpu.VMEM((1024,), jnp.int32),
            pltpu.VMEM((1024,), jnp.int32),
            pltpu.VMEM((1024, HALF), jnp.bfloat16),
            pltpu.VMEM((1024,), jnp.int32),
            pltpu.VMEM((1024,), jnp.int32),
            pltpu.VMEM((1024, HALF), jnp.bfloat16),
            pltpu.VMEM((1024, HALF), jnp.bfloat16),
            pltpu.VMEM((1024, HALF), jnp.bfloat16),
            pltpu.VMEM((BR, HALF), jnp.bfloat16),
            pltpu.VMEM((BR, HALF), jnp.bfloat16),
            pltpu.VMEM((BR, HALF), jnp.bfloat16),
            pltpu.VMEM_SHARED((NN, HALF), jnp.bfloat16),
            pltpu.SemaphoreType.DMA,
            pltpu.SemaphoreType.DMA,
            pltpu.SemaphoreType.DMA,
            pltpu.SemaphoreType.DMA,
            pltpu.SemaphoreType.DMA,
            pltpu.SemaphoreType.DMA,
        ],
    )
    def k(z0_hbm, z1_hbm, src_hbm, dst_hbm, w_hbm, l0_hbm, l1_hbm,
          sb0, db0, wb0, sb1, db1, wb1, rows0, rows1, ba, bb, zb, acc,
          gs0, gs1, ss0, ss1, ws0, ws1):
        c = lax.axis_index("c")
        s = lax.axis_index("s")
        base = s * KTILE

        @pl.loop(0, BR)
        def _zzb(r):
            zb[r, :] = jnp.zeros((HALF,), jnp.bfloat16)

        def load_idx(cidx, sbx, dbx, wbx, wsx):
            eoff = cidx * 1024
            pltpu.async_copy(w_hbm.at[pl.ds(eoff, 1024)], wbx, wsx)
            pltpu.sync_copy(src_hbm.at[pl.ds(eoff, 1024)], sbx)
            pltpu.sync_copy(dst_hbm.at[pl.ds(eoff, 1024)], dbx)

        def wait_w(wbx, wsx):
            pltpu.make_async_copy(
                w_hbm.at[pl.ds(0, 1024)], wbx, wsx).wait()

        def fire_gathers(l_ref, sbx, rowsx, gsx):
            pltpu.async_copy(l_ref.at[sbx], rowsx, gsx)

        def fire_scatters(rowsx, dbx, ssx):
            pltpu.async_copy(rowsx, acc.at[dbx], ssx, add=True)

        def drain(z_ref, bufx, semx):
            pltpu.make_async_copy(z_ref.at[pl.ds(0, 1024)], bufx, semx).wait()

        def scale(rowsx, wbx):
            @pl.loop(0, 1024, step=8)
            def _scale(e0):
                for i in range(8):
                    rowsx[e0 + i, :] = rowsx[e0 + i, :] * wbx[e0 + i, :]

        def work(z_ref, l_ref):
            @pl.loop(s, NN // BR, step=NSUB)
            def _pro(kk):
                r0 = kk * BR
                pltpu.sync_copy(z_ref.at[pl.ds(r0, BR)], ba)
                pltpu.sync_copy(ba, l_ref.at[pl.ds(r0, BR)])
                pltpu.sync_copy(zb, acc.at[pl.ds(r0, BR)])

            plsc.subcore_barrier()

            @pl.loop(0, IT)
            def _iter(it):
                load_idx(base, sb0, db0, wb0, ws0)
                fire_gathers(l_ref, sb0, rows0, gs0)
                load_idx(base + 1, sb1, db1, wb1, ws1)
                fire_gathers(l_ref, sb1, rows1, gs1)

                @pl.loop(0, KTILE // 2)
                def _pair(g):
                    c0 = base + 2 * g
                    drain(z_ref, rows0, gs0)
                    wait_w(wb0, ws0)
                    scale(rows0, wb0)
                    fire_scatters(rows0, db0, ss0)
                    drain(z_ref, rows1, gs1)
                    wait_w(wb1, ws1)
                    scale(rows1, wb1)
                    fire_scatters(rows1, db1, ss1)

                    @pl.when(g < KTILE // 2 - 1)
                    def _prep():
                        drain(z_ref, rows0, ss0)
                        load_idx(c0 + 2, sb0, db0, wb0, ws0)
                        fire_gathers(l_ref, sb0, rows0, gs0)
                        drain(z_ref, rows1, ss1)
                        load_idx(c0 + 3, sb1, db1, wb1, ws1)
                        fire_gathers(l_ref, sb1, rows1, gs1)

                drain(z_ref, rows0, ss0)
                drain(z_ref, rows1, ss1)

                plsc.subcore_barrier()

                @pl.loop(s, NN // BR, step=NSUB)
                def _blend(kk):
                    r0 = kk * BR
                    pltpu.sync_copy(acc.at[pl.ds(r0, BR)], ba)
                    pltpu.sync_copy(z_ref.at[pl.ds(r0, BR)], bb)

                    @pl.loop(0, BR)
                    def _rowb(r):
                        ba[r, :] = (1.0 - AL) * ba[r, :] + AL * bb[r, :]

                    pltpu.sync_copy(ba, l_ref.at[pl.ds(r0, BR)])
                    pltpu.sync_copy(zb, acc.at[pl.ds(r0, BR)])

                plsc.subcore_barrier()

        @pl.when(c == 0)
        def _():
            work(z0_hbm, l0_hbm)

        @pl.when(c == 1)
        def _():
            work(z1_hbm, l1_hbm)

    return k(z0, z1, src, dst, w)


def _log_softmax(l0, l1):
    bm = 2000
    grid = (NN // bm,)

    def body(l0_ref, l1_ref, o_ref):
        x = jnp.concatenate(
            [l0_ref[...], l1_ref[...]], axis=1).astype(jnp.float32)
        m = jnp.max(x, axis=1, keepdims=True)
        xm = x - m
        lse = jnp.log(jnp.sum(jnp.exp(xm), axis=1, keepdims=True))
        o_ref[...] = xm - lse

    return pl.pallas_call(
        body,
        grid=grid,
        in_specs=[
            pl.BlockSpec((bm, HALF), lambda i: (i, 0)),
            pl.BlockSpec((bm, HALF), lambda i: (i, 0)),
        ],
        out_specs=pl.BlockSpec((bm, LL), lambda i: (i, 0)),
        out_shape=jax.ShapeDtypeStruct((NN, LL), jnp.float32),
    )(l0, l1)


def kernel(feature_indices, feature_values, edge_indices, edge_weights,
           W1, b1, W2, b2):
    frow = feature_indices[0].astype(jnp.int32)
    fcol = feature_indices[1].astype(jnp.int32)
    fval = feature_values.astype(jnp.float32)
    fpad = F_ROWS * 128 - ZF
    frow = jnp.concatenate([frow, jnp.zeros((fpad,), jnp.int32)]).reshape(
        F_ROWS, 128)
    fcol = jnp.concatenate([fcol, jnp.zeros((fpad,), jnp.int32)]).reshape(
        F_ROWS, 128)
    fval = jnp.concatenate([fval, jnp.zeros((fpad,), jnp.float32)]).reshape(
        F_ROWS, 128)

    src = edge_indices[1].astype(jnp.int32)
    dst = edge_indices[0].astype(jnp.int32)
    wgt = edge_weights.astype(jnp.float32)
    epad = E_ROWS * 128 - EE
    src = jnp.concatenate([src, jnp.zeros((epad,), jnp.int32)])
    dst = jnp.concatenate([dst, jnp.zeros((epad,), jnp.int32)])
    wgt = jnp.concatenate([wgt, jnp.zeros((epad,), jnp.float32)])
    wgt = jnp.broadcast_to(
        wgt.astype(jnp.bfloat16)[:, None], (E_ROWS * 128, HALF))

    x = _densify(frow, fcol, fval).reshape(NN, FF)
    z0, z1 = _mlp(x, W1, b1.reshape(1, HH), W2, b2.reshape(1, LL))
    l0, l1 = _propagate(z0, z1, src, dst, wgt)
    return _log_softmax(l0, l1)


# speedup vs baseline: 10.9534x; 1.1400x over previous
import dataclasses
import functools

import jax
import jax.numpy as jnp
from jax import lax
from jax.experimental import pallas as pl
from jax.experimental.pallas import tpu as pltpu
from jax.experimental.pallas import tpu_sc as plsc

NN = 50000
FF = 128
HH = 128
LL = 64
EE = 800000
ZF = 1000000
AL = 0.1
IT = 10

NSUB = 16
HALF = LL // 2
QROWS = NN // 4
QFLAT = QROWS * FF

F_ROWS = 7824
E_ROWS = 6400
F_CHUNKS = F_ROWS // 16
CH = 320
E_CHUNKS = E_ROWS * 128 // CH
KTILE = E_CHUNKS // NSUB

BR = 80


def _sc_mesh():
    return plsc.VectorSubcoreMesh(core_axis_name="c", subcore_axis_name="s")


def _sc_params():
    cp = pltpu.CompilerParams()
    if "needs_layout_passes" in pltpu.CompilerParams.__dataclass_fields__:
        cp = dataclasses.replace(cp, needs_layout_passes=False)
    if "use_tc_tiling_on_sc" in pltpu.CompilerParams.__dataclass_fields__:
        cp = dataclasses.replace(cp, use_tc_tiling_on_sc=False)
    return cp


def _densify(frow, fcol, fval):
    @functools.partial(
        pl.kernel,
        out_type=jax.ShapeDtypeStruct((NN * FF,), jnp.float32),
        mesh=_sc_mesh(),
        scratch_types=[
            pltpu.VMEM((16, 128), jnp.int32),
            pltpu.VMEM((16, 128), jnp.int32),
            pltpu.VMEM((16, 128), jnp.float32),
            pltpu.VMEM((10000,), jnp.float32),
            pltpu.VMEM((10000,), jnp.float32),
            pltpu.VMEM_SHARED((QFLAT + 128,), jnp.float32),
            pltpu.SemaphoreType.DMA,
        ],
    )
    def k(frow_hbm, fcol_hbm, fval_hbm, x_hbm, rb, cb, vb, stage, stage2,
          acc, sem):
        c = lax.axis_index("c")
        s = lax.axis_index("s")

        @pl.loop(0, 10000, step=16)
        def _zstage(i):
            stage[pl.ds(i, 16)] = jnp.zeros((16,), jnp.float32)

        @pl.loop(0, 2)
        def _pass(p):
            q = 2 * c + p
            base = q * QROWS

            @pl.loop(0, 10)
            def _z(i):
                pltpu.sync_copy(
                    stage, acc.at[pl.ds(s * 100000 + i * 10000, 10000)])

            plsc.subcore_barrier()

            @pl.loop(s, F_CHUNKS, step=NSUB)
            def _chunk(j):
                roff = j * 16
                pltpu.sync_copy(frow_hbm.at[pl.ds(roff, 16)], rb)
                pltpu.sync_copy(fcol_hbm.at[pl.ds(roff, 16)], cb)
                pltpu.sync_copy(fval_hbm.at[pl.ds(roff, 16)], vb)

                @pl.loop(0, 16)
                def _row(i):
                    @pl.loop(0, 128, step=16)
                    def _lane(l):
                        r = rb[i, pl.ds(l, 16)]
                        cc = cb[i, pl.ds(l, 16)]
                        rel = r - base
                        ok = (rel >= 0) & (rel < QROWS)
                        rb[i, pl.ds(l, 16)] = jnp.where(
                            ok, rel * FF + cc, QFLAT + cc)

                for t in range(16):
                    pltpu.sync_copy(vb.at[t], acc.at[rb.at[t]], add=True)

            plsc.subcore_barrier()

            @pl.loop(0, 10)
            def _out(i):
                off = s * 100000 + i * 10000
                pltpu.sync_copy(acc.at[pl.ds(off, 10000)], stage2)
                pltpu.sync_copy(stage2,
                                x_hbm.at[pl.ds(q * QFLAT + off, 10000)])

    return k(frow, fcol, fval)


def _mlp(x, w1, b1, w2, b2):
    bm = 2000
    grid = (NN // bm,)

    def body(x_ref, w1_ref, b1_ref, w2_ref, b2_ref, z0_ref, z1_ref):
        h = lax.dot_general(
            x_ref[...].astype(jnp.float32), w1_ref[...],
            (((1,), (0,)), ((), ())),
            precision=lax.Precision.HIGHEST,
            preferred_element_type=jnp.float32)
        h = jnp.maximum(h + b1_ref[...], 0.0)
        z = lax.dot_general(
            h, w2_ref[...], (((1,), (0,)), ((), ())),
            precision=lax.Precision.HIGHEST,
            preferred_element_type=jnp.float32) + b2_ref[...]
        z0_ref[...] = z[:, :HALF].astype(jnp.bfloat16)
        z1_ref[...] = z[:, HALF:].astype(jnp.bfloat16)

    return pl.pallas_call(
        body,
        grid=grid,
        in_specs=[
            pl.BlockSpec((bm, FF), lambda i: (i, 0)),
            pl.BlockSpec((FF, HH), lambda i: (0, 0)),
            pl.BlockSpec((1, HH), lambda i: (0, 0)),
            pl.BlockSpec((HH, LL), lambda i: (0, 0)),
            pl.BlockSpec((1, LL), lambda i: (0, 0)),
        ],
        out_specs=[
            pl.BlockSpec((bm, HALF), lambda i: (i, 0)),
            pl.BlockSpec((bm, HALF), lambda i: (i, 0)),
        ],
        out_shape=[
            jax.ShapeDtypeStruct((NN, HALF), jnp.bfloat16),
            jax.ShapeDtypeStruct((NN, HALF), jnp.bfloat16),
        ],
    )(x, w1, b1, w2, b2)


def _propagate(z0, z1, src, dst, w):
    @functools.partial(
        pl.kernel,
        out_type=(
            jax.ShapeDtypeStruct((NN, HALF), jnp.bfloat16),
            jax.ShapeDtypeStruct((NN, HALF), jnp.bfloat16),
        ),
        mesh=_sc_mesh(),
        compiler_params=_sc_params(),
        scratch_types=[
            pltpu.VMEM((CH,), jnp.int32),
            pltpu.VMEM((CH,), jnp.int32),
            pltpu.VMEM((CH, HALF), jnp.bfloat16),
            pltpu.VMEM((CH,), jnp.int32),
            pltpu.VMEM((CH,), jnp.int32),
            pltpu.VMEM((CH, HALF), jnp.bfloat16),
            pltpu.VMEM((CH, HALF), jnp.bfloat16),
            pltpu.VMEM((CH, HALF), jnp.bfloat16),
            pltpu.VMEM((BR, HALF), jnp.bfloat16),
            pltpu.VMEM((BR, HALF), jnp.bfloat16),
            pltpu.VMEM((BR, HALF), jnp.bfloat16),
            pltpu.VMEM_SHARED((NN, HALF), jnp.bfloat16),
            pltpu.VMEM_SHARED((NN, HALF), jnp.bfloat16),
            pltpu.SemaphoreType.DMA,
            pltpu.SemaphoreType.DMA,
            pltpu.SemaphoreType.DMA,
            pltpu.SemaphoreType.DMA,
            pltpu.SemaphoreType.DMA,
            pltpu.SemaphoreType.DMA,
        ],
    )
    def k(z0_hbm, z1_hbm, src_hbm, dst_hbm, w_hbm, l0_hbm, l1_hbm,
          sb0, db0, wb0, sb1, db1, wb1, rows0, rows1, ba, bb, zb, acc, loc,
          gs0, gs1, ss0, ss1, ws0, ws1):
        c = lax.axis_index("c")
        s = lax.axis_index("s")
        base = s * KTILE

        @pl.loop(0, BR)
        def _zzb(r):
            zb[r, :] = jnp.zeros((HALF,), jnp.bfloat16)

        def load_idx(cidx, sbx, dbx, wbx, wsx):
            eoff = cidx * CH
            pltpu.async_copy(w_hbm.at[pl.ds(eoff, CH)], wbx, wsx)
            pltpu.sync_copy(src_hbm.at[pl.ds(eoff, CH)], sbx)
            pltpu.sync_copy(dst_hbm.at[pl.ds(eoff, CH)], dbx)

        def wait_w(wbx, wsx):
            pltpu.make_async_copy(
                w_hbm.at[pl.ds(0, CH)], wbx, wsx).wait()

        def fire_gathers(sbx, rowsx, gsx):
            pltpu.async_copy(loc.at[sbx], rowsx, gsx)

        def fire_scatters(rowsx, dbx, ssx):
            pltpu.async_copy(rowsx, acc.at[dbx], ssx, add=True)

        def drain(z_ref, bufx, semx):
            pltpu.make_async_copy(z_ref.at[pl.ds(0, CH)], bufx, semx).wait()

        def scale(rowsx, wbx):
            @pl.loop(0, CH, step=8)
            def _scale(e0):
                for i in range(8):
                    rowsx[e0 + i, :] = rowsx[e0 + i, :] * wbx[e0 + i, :]

        def work(z_ref, l_ref):
            @pl.loop(s, NN // BR, step=NSUB)
            def _pro(kk):
                r0 = kk * BR
                pltpu.sync_copy(z_ref.at[pl.ds(r0, BR)], loc.at[pl.ds(r0, BR)])
                pltpu.sync_copy(zb, acc.at[pl.ds(r0, BR)])

            plsc.subcore_barrier()

            @pl.loop(0, IT)
            def _iter(it):
                load_idx(base, sb0, db0, wb0, ws0)
                fire_gathers(sb0, rows0, gs0)
                load_idx(base + 1, sb1, db1, wb1, ws1)
                fire_gathers(sb1, rows1, gs1)

                @pl.loop(0, KTILE // 2)
                def _pair(g):
                    c0 = base + 2 * g
                    drain(z_ref, rows0, gs0)
                    wait_w(wb0, ws0)
                    scale(rows0, wb0)
                    fire_scatters(rows0, db0, ss0)
                    drain(z_ref, rows1, gs1)
                    wait_w(wb1, ws1)
                    scale(rows1, wb1)
                    fire_scatters(rows1, db1, ss1)

                    @pl.when(g < KTILE // 2 - 1)
                    def _prep():
                        drain(z_ref, rows0, ss0)
                        load_idx(c0 + 2, sb0, db0, wb0, ws0)
                        fire_gathers(sb0, rows0, gs0)
                        drain(z_ref, rows1, ss1)
                        load_idx(c0 + 3, sb1, db1, wb1, ws1)
                        fire_gathers(sb1, rows1, gs1)

                drain(z_ref, rows0, ss0)
                drain(z_ref, rows1, ss1)

                plsc.subcore_barrier()

                @pl.loop(s, NN // BR, step=NSUB)
                def _blend(kk):
                    r0 = kk * BR
                    pltpu.sync_copy(acc.at[pl.ds(r0, BR)], ba)
                    pltpu.sync_copy(z_ref.at[pl.ds(r0, BR)], bb)

                    @pl.loop(0, BR)
                    def _rowb(r):
                        ba[r, :] = (1.0 - AL) * ba[r, :] + AL * bb[r, :]

                    pltpu.sync_copy(ba, loc.at[pl.ds(r0, BR)])
                    pltpu.sync_copy(zb, acc.at[pl.ds(r0, BR)])

                    @pl.when(it == IT - 1)
                    def _emit():
                        pltpu.sync_copy(ba, l_ref.at[pl.ds(r0, BR)])

                plsc.subcore_barrier()

        @pl.when(c == 0)
        def _():
            work(z0_hbm, l0_hbm)

        @pl.when(c == 1)
        def _():
            work(z1_hbm, l1_hbm)

    return k(z0, z1, src, dst, w)


def _log_softmax(l0, l1):
    bm = 2000
    grid = (NN // bm,)

    def body(l0_ref, l1_ref, o_ref):
        x = jnp.concatenate(
            [l0_ref[...], l1_ref[...]], axis=1).astype(jnp.float32)
        m = jnp.max(x, axis=1, keepdims=True)
        xm = x - m
        lse = jnp.log(jnp.sum(jnp.exp(xm), axis=1, keepdims=True))
        o_ref[...] = xm - lse

    return pl.pallas_call(
        body,
        grid=grid,
        in_specs=[
            pl.BlockSpec((bm, HALF), lambda i: (i, 0)),
            pl.BlockSpec((bm, HALF), lambda i: (i, 0)),
        ],
        out_specs=pl.BlockSpec((bm, LL), lambda i: (i, 0)),
        out_shape=jax.ShapeDtypeStruct((NN, LL), jnp.float32),
    )(l0, l1)


def kernel(feature_indices, feature_values, edge_indices, edge_weights,
           W1, b1, W2, b2):
    frow = feature_indices[0].astype(jnp.int32)
    fcol = feature_indices[1].astype(jnp.int32)
    fval = feature_values.astype(jnp.float32)
    fpad = F_ROWS * 128 - ZF
    frow = jnp.concatenate([frow, jnp.zeros((fpad,), jnp.int32)]).reshape(
        F_ROWS, 128)
    fcol = jnp.concatenate([fcol, jnp.zeros((fpad,), jnp.int32)]).reshape(
        F_ROWS, 128)
    fval = jnp.concatenate([fval, jnp.zeros((fpad,), jnp.float32)]).reshape(
        F_ROWS, 128)

    src = edge_indices[1].astype(jnp.int32)
    dst = edge_indices[0].astype(jnp.int32)
    wgt = edge_weights.astype(jnp.float32)
    epad = E_ROWS * 128 - EE
    src = jnp.concatenate([src, jnp.zeros((epad,), jnp.int32)])
    dst = jnp.concatenate([dst, jnp.zeros((epad,), jnp.int32)])
    wgt = jnp.concatenate([wgt, jnp.zeros((epad,), jnp.float32)])
    wgt = jnp.broadcast_to(
        wgt.astype(jnp.bfloat16)[:, None], (E_ROWS * 128, HALF))

    x = _densify(frow, fcol, fval).reshape(NN, FF)
    z0, z1 = _mlp(x, W1, b1.reshape(1, HH), W2, b2.reshape(1, LL))
    l0, l1 = _propagate(z0, z1, src, dst, wgt)
    return _log_softmax(l0, l1)


# speedup vs baseline: 14.5230x; 1.3259x over previous
import dataclasses
import functools

import jax
import jax.numpy as jnp
from jax import lax
from jax.experimental import pallas as pl
from jax.experimental.pallas import tpu as pltpu
from jax.experimental.pallas import tpu_sc as plsc

NN = 50000
FF = 128
HH = 128
LL = 64
EE = 800000
ZF = 1000000
AL = 0.1
IT = 10

NSUB = 16
HALF = LL // 2
QROWS = NN // 4
QFLAT = QROWS * FF

F_ROWS = 7824
E_ROWS = 6400
F_CHUNKS = F_ROWS // 16
CH = 320
E_CHUNKS = E_ROWS * 128 // CH
KTILE = E_CHUNKS // NSUB

BR = 80


def _sc_mesh():
    return plsc.VectorSubcoreMesh(core_axis_name="c", subcore_axis_name="s")


def _sc_params():
    cp = pltpu.CompilerParams()
    if "needs_layout_passes" in pltpu.CompilerParams.__dataclass_fields__:
        cp = dataclasses.replace(cp, needs_layout_passes=False)
    if "use_tc_tiling_on_sc" in pltpu.CompilerParams.__dataclass_fields__:
        cp = dataclasses.replace(cp, use_tc_tiling_on_sc=False)
    return cp


def _densify(frow, fcol, fval):
    @functools.partial(
        pl.kernel,
        out_type=jax.ShapeDtypeStruct((NN * FF,), jnp.float32),
        mesh=_sc_mesh(),
        scratch_types=[
            pltpu.VMEM((16, 128), jnp.int32),
            pltpu.VMEM((16, 128), jnp.int32),
            pltpu.VMEM((16, 128), jnp.float32),
            pltpu.VMEM((10000,), jnp.float32),
            pltpu.VMEM((10000,), jnp.float32),
            pltpu.VMEM_SHARED((QFLAT + 128,), jnp.float32),
            pltpu.SemaphoreType.DMA,
        ],
    )
    def k(frow_hbm, fcol_hbm, fval_hbm, x_hbm, rb, cb, vb, stage, stage2,
          acc, sem):
        c = lax.axis_index("c")
        s = lax.axis_index("s")

        @pl.loop(0, 10000, step=16)
        def _zstage(i):
            stage[pl.ds(i, 16)] = jnp.zeros((16,), jnp.float32)

        @pl.loop(0, 2)
        def _pass(p):
            q = 2 * c + p
            base = q * QROWS

            @pl.loop(0, 10)
            def _z(i):
                pltpu.sync_copy(
                    stage, acc.at[pl.ds(s * 100000 + i * 10000, 10000)])

            plsc.subcore_barrier()

            @pl.loop(s, F_CHUNKS, step=NSUB)
            def _chunk(j):
                roff = j * 16
                pltpu.sync_copy(frow_hbm.at[pl.ds(roff, 16)], rb)
                pltpu.sync_copy(fcol_hbm.at[pl.ds(roff, 16)], cb)
                pltpu.sync_copy(fval_hbm.at[pl.ds(roff, 16)], vb)

                @pl.loop(0, 16)
                def _row(i):
                    @pl.loop(0, 128, step=16)
                    def _lane(l):
                        r = rb[i, pl.ds(l, 16)]
                        cc = cb[i, pl.ds(l, 16)]
                        rel = r - base
                        ok = (rel >= 0) & (rel < QROWS)
                        rb[i, pl.ds(l, 16)] = jnp.where(
                            ok, rel * FF + cc, QFLAT + cc)

                for t in range(16):
                    pltpu.sync_copy(vb.at[t], acc.at[rb.at[t]], add=True)

            plsc.subcore_barrier()

            @pl.loop(0, 10)
            def _out(i):
                off = s * 100000 + i * 10000
                pltpu.sync_copy(acc.at[pl.ds(off, 10000)], stage2)
                pltpu.sync_copy(stage2,
                                x_hbm.at[pl.ds(q * QFLAT + off, 10000)])

    return k(frow, fcol, fval)


def _mlp(x, w1, b1, w2, b2):
    bm = 2000
    grid = (NN // bm,)

    def body(x_ref, w1_ref, b1_ref, w2_ref, b2_ref, z0_ref, z1_ref):
        h = lax.dot_general(
            x_ref[...].astype(jnp.float32), w1_ref[...],
            (((1,), (0,)), ((), ())),
            precision=lax.Precision.HIGHEST,
            preferred_element_type=jnp.float32)
        h = jnp.maximum(h + b1_ref[...], 0.0)
        z = lax.dot_general(
            h, w2_ref[...], (((1,), (0,)), ((), ())),
            precision=lax.Precision.HIGHEST,
            preferred_element_type=jnp.float32) + b2_ref[...]
        z0_ref[...] = z[:, :HALF].astype(jnp.bfloat16)
        z1_ref[...] = z[:, HALF:].astype(jnp.bfloat16)

    return pl.pallas_call(
        body,
        grid=grid,
        in_specs=[
            pl.BlockSpec((bm, FF), lambda i: (i, 0)),
            pl.BlockSpec((FF, HH), lambda i: (0, 0)),
            pl.BlockSpec((1, HH), lambda i: (0, 0)),
            pl.BlockSpec((HH, LL), lambda i: (0, 0)),
            pl.BlockSpec((1, LL), lambda i: (0, 0)),
        ],
        out_specs=[
            pl.BlockSpec((bm, HALF), lambda i: (i, 0)),
            pl.BlockSpec((bm, HALF), lambda i: (i, 0)),
        ],
        out_shape=[
            jax.ShapeDtypeStruct((NN, HALF), jnp.bfloat16),
            jax.ShapeDtypeStruct((NN, HALF), jnp.bfloat16),
        ],
    )(x, w1, b1, w2, b2)


def _propagate(z0, z1, src, dst, w):
    @functools.partial(
        pl.kernel,
        out_type=(
            jax.ShapeDtypeStruct((NN, HALF), jnp.bfloat16),
            jax.ShapeDtypeStruct((NN, HALF), jnp.bfloat16),
        ),
        mesh=_sc_mesh(),
        compiler_params=_sc_params(),
        scratch_types=[
            pltpu.VMEM((CH,), jnp.int32),
            pltpu.VMEM((CH,), jnp.int32),
            pltpu.VMEM((CH, HALF), jnp.bfloat16),
            pltpu.VMEM((CH,), jnp.int32),
            pltpu.VMEM((CH,), jnp.int32),
            pltpu.VMEM((CH, HALF), jnp.bfloat16),
            pltpu.VMEM((CH, HALF), jnp.bfloat16),
            pltpu.VMEM((CH, HALF), jnp.bfloat16),
            pltpu.VMEM((CH,), jnp.int32),
            pltpu.VMEM((CH,), jnp.int32),
            pltpu.VMEM((BR, HALF), jnp.bfloat16),
            pltpu.VMEM((BR, HALF), jnp.bfloat16),
            pltpu.VMEM((BR, HALF), jnp.bfloat16),
            pltpu.VMEM_SHARED((NN, HALF), jnp.bfloat16),
            pltpu.VMEM_SHARED((NN, HALF), jnp.bfloat16),
            pltpu.SemaphoreType.DMA,
            pltpu.SemaphoreType.DMA,
            pltpu.SemaphoreType.DMA,
            pltpu.SemaphoreType.DMA,
            pltpu.SemaphoreType.DMA,
            pltpu.SemaphoreType.DMA,
            pltpu.SemaphoreType.DMA,
            pltpu.SemaphoreType.DMA,
        ],
    )
    def k(z0_hbm, z1_hbm, src_hbm, dst_hbm, w_hbm, l0_hbm, l1_hbm,
          sb0, db0, wb0, sb1, db1, wb1, rows0, rows1, dbf0, dbf1,
          ba, bb, zb, acc, loc,
          gs0, gs1, ss0, ss1, ws0, ws1, is0, is1):
        c = lax.axis_index("c")
        s = lax.axis_index("s")
        base = s * KTILE

        @pl.loop(0, BR)
        def _zzb(r):
            zb[r, :] = jnp.zeros((HALF,), jnp.bfloat16)

        def load_idx(cidx, sbx, dbx, wbx, wsx, isx):
            eoff = cidx * CH
            pltpu.async_copy(w_hbm.at[pl.ds(eoff, CH)], wbx, wsx)
            pltpu.async_copy(src_hbm.at[pl.ds(eoff, CH)], sbx, isx)
            pltpu.async_copy(dst_hbm.at[pl.ds(eoff, CH)], dbx, isx)

        def wait_w(wbx, wsx):
            pltpu.make_async_copy(
                w_hbm.at[pl.ds(0, CH)], wbx, wsx).wait()

        def wait_idx(sbx, dbx, isx):
            pltpu.make_async_copy(
                src_hbm.at[pl.ds(0, CH)], sbx, isx).wait()
            pltpu.make_async_copy(
                dst_hbm.at[pl.ds(0, CH)], dbx, isx).wait()

        def cpidx(srcb, dstb):
            @pl.loop(0, CH, step=16)
            def _c(i):
                dstb[pl.ds(i, 16)] = srcb[pl.ds(i, 16)]

        def fire_gathers(sbx, rowsx, gsx):
            pltpu.async_copy(loc.at[sbx], rowsx, gsx)

        def fire_scatters(rowsx, dbx, ssx):
            pltpu.async_copy(rowsx, acc.at[dbx], ssx, add=True)

        def drain(z_ref, bufx, semx):
            pltpu.make_async_copy(z_ref.at[pl.ds(0, CH)], bufx, semx).wait()

        def scale(rowsx, wbx):
            @pl.loop(0, CH, step=8)
            def _scale(e0):
                for i in range(8):
                    rowsx[e0 + i, :] = rowsx[e0 + i, :] * wbx[e0 + i, :]

        def work(z_ref, l_ref):
            @pl.loop(s, NN // BR, step=NSUB)
            def _pro(kk):
                r0 = kk * BR
                pltpu.sync_copy(z_ref.at[pl.ds(r0, BR)], loc.at[pl.ds(r0, BR)])
                pltpu.sync_copy(zb, acc.at[pl.ds(r0, BR)])

            plsc.subcore_barrier()

            @pl.loop(0, IT)
            def _iter(it):
                load_idx(base, sb0, db0, wb0, ws0, is0)
                wait_idx(sb0, db0, is0)
                cpidx(db0, dbf0)
                fire_gathers(sb0, rows0, gs0)
                load_idx(base + 1, sb1, db1, wb1, ws1, is1)
                wait_idx(sb1, db1, is1)
                cpidx(db1, dbf1)
                fire_gathers(sb1, rows1, gs1)

                @pl.loop(0, KTILE // 2)
                def _pair(g):
                    c0 = base + 2 * g
                    drain(z_ref, rows0, gs0)
                    wait_w(wb0, ws0)
                    scale(rows0, wb0)
                    fire_scatters(rows0, dbf0, ss0)

                    @pl.when(g < KTILE // 2 - 1)
                    def _pf0():
                        load_idx(c0 + 2, sb0, db0, wb0, ws0, is0)

                    drain(z_ref, rows1, gs1)
                    wait_w(wb1, ws1)
                    scale(rows1, wb1)
                    fire_scatters(rows1, dbf1, ss1)

                    @pl.when(g < KTILE // 2 - 1)
                    def _pf1():
                        load_idx(c0 + 3, sb1, db1, wb1, ws1, is1)

                    @pl.when(g < KTILE // 2 - 1)
                    def _prep():
                        drain(z_ref, rows0, ss0)
                        wait_idx(sb0, db0, is0)
                        cpidx(db0, dbf0)
                        fire_gathers(sb0, rows0, gs0)
                        drain(z_ref, rows1, ss1)
                        wait_idx(sb1, db1, is1)
                        cpidx(db1, dbf1)
                        fire_gathers(sb1, rows1, gs1)

                drain(z_ref, rows0, ss0)
                drain(z_ref, rows1, ss1)

                plsc.subcore_barrier()

                @pl.loop(s, NN // BR, step=NSUB)
                def _blend(kk):
                    r0 = kk * BR
                    pltpu.sync_copy(acc.at[pl.ds(r0, BR)], ba)
                    pltpu.sync_copy(z_ref.at[pl.ds(r0, BR)], bb)

                    @pl.loop(0, BR)
                    def _rowb(r):
                        ba[r, :] = (1.0 - AL) * ba[r, :] + AL * bb[r, :]

                    pltpu.sync_copy(ba, loc.at[pl.ds(r0, BR)])
                    pltpu.sync_copy(zb, acc.at[pl.ds(r0, BR)])

                    @pl.when(it == IT - 1)
                    def _emit():
                        pltpu.sync_copy(ba, l_ref.at[pl.ds(r0, BR)])

                plsc.subcore_barrier()

        @pl.when(c == 0)
        def _():
            work(z0_hbm, l0_hbm)

        @pl.when(c == 1)
        def _():
            work(z1_hbm, l1_hbm)

    return k(z0, z1, src, dst, w)


def _log_softmax(l0, l1):
    bm = 2000
    grid = (NN // bm,)

    def body(l0_ref, l1_ref, o_ref):
        x = jnp.concatenate(
            [l0_ref[...], l1_ref[...]], axis=1).astype(jnp.float32)
        m = jnp.max(x, axis=1, keepdims=True)
        xm = x - m
        lse = jnp.log(jnp.sum(jnp.exp(xm), axis=1, keepdims=True))
        o_ref[...] = xm - lse

    return pl.pallas_call(
        body,
        grid=grid,
        in_specs=[
            pl.BlockSpec((bm, HALF), lambda i: (i, 0)),
            pl.BlockSpec((bm, HALF), lambda i: (i, 0)),
        ],
        out_specs=pl.BlockSpec((bm, LL), lambda i: (i, 0)),
        out_shape=jax.ShapeDtypeStruct((NN, LL), jnp.float32),
    )(l0, l1)


def kernel(feature_indices, feature_values, edge_indices, edge_weights,
           W1, b1, W2, b2):
    frow = feature_indices[0].astype(jnp.int32)
    fcol = feature_indices[1].astype(jnp.int32)
    fval = feature_values.astype(jnp.float32)
    fpad = F_ROWS * 128 - ZF
    frow = jnp.concatenate([frow, jnp.zeros((fpad,), jnp.int32)]).reshape(
        F_ROWS, 128)
    fcol = jnp.concatenate([fcol, jnp.zeros((fpad,), jnp.int32)]).reshape(
        F_ROWS, 128)
    fval = jnp.concatenate([fval, jnp.zeros((fpad,), jnp.float32)]).reshape(
        F_ROWS, 128)

    src = edge_indices[1].astype(jnp.int32)
    dst = edge_indices[0].astype(jnp.int32)
    wgt = edge_weights.astype(jnp.float32)
    epad = E_ROWS * 128 - EE
    src = jnp.concatenate([src, jnp.zeros((epad,), jnp.int32)])
    dst = jnp.concatenate([dst, jnp.zeros((epad,), jnp.int32)])
    wgt = jnp.concatenate([wgt, jnp.zeros((epad,), jnp.float32)])
    wgt = jnp.broadcast_to(
        wgt.astype(jnp.bfloat16)[:, None], (E_ROWS * 128, HALF))

    x = _densify(frow, fcol, fval).reshape(NN, FF)
    z0, z1 = _mlp(x, W1, b1.reshape(1, HH), W2, b2.reshape(1, LL))
    l0, l1 = _propagate(z0, z1, src, dst, wgt)
    return _log_softmax(l0, l1)


# speedup vs baseline: 15.4451x; 1.0635x over previous
import dataclasses
import functools

import jax
import jax.numpy as jnp
from jax import lax
from jax.experimental import pallas as pl
from jax.experimental.pallas import tpu as pltpu
from jax.experimental.pallas import tpu_sc as plsc

NN = 50000
FF = 128
HH = 128
LL = 64
EE = 800000
ZF = 1000000
AL = 0.1
IT = 10

NSUB = 16
HALF = LL // 2
QROWS = NN // 4
QFLAT = QROWS * FF

F_ROWS = 7824
E_ROWS = 6400
F_CHUNKS = F_ROWS // 16
CH = 320
E_CHUNKS = E_ROWS * 128 // CH
KTILE = E_CHUNKS // NSUB

BR = 200
BZ = 100


def _sc_mesh():
    return plsc.VectorSubcoreMesh(core_axis_name="c", subcore_axis_name="s")


def _sc_params():
    cp = pltpu.CompilerParams()
    if "needs_layout_passes" in pltpu.CompilerParams.__dataclass_fields__:
        cp = dataclasses.replace(cp, needs_layout_passes=False)
    if "use_tc_tiling_on_sc" in pltpu.CompilerParams.__dataclass_fields__:
        cp = dataclasses.replace(cp, use_tc_tiling_on_sc=False)
    return cp


def _densify(frow, fcol, fval):
    @functools.partial(
        pl.kernel,
        out_type=jax.ShapeDtypeStruct((NN * FF,), jnp.float32),
        mesh=_sc_mesh(),
        scratch_types=[
            pltpu.VMEM((16, 128), jnp.int32),
            pltpu.VMEM((16, 128), jnp.int32),
            pltpu.VMEM((16, 128), jnp.float32),
            pltpu.VMEM((10000,), jnp.float32),
            pltpu.VMEM((10000,), jnp.float32),
            pltpu.VMEM_SHARED((QFLAT + 128,), jnp.float32),
            pltpu.SemaphoreType.DMA,
        ],
    )
    def k(frow_hbm, fcol_hbm, fval_hbm, x_hbm, rb, cb, vb, stage, stage2,
          acc, sem):
        c = lax.axis_index("c")
        s = lax.axis_index("s")

        @pl.loop(0, 10000, step=16)
        def _zstage(i):
            stage[pl.ds(i, 16)] = jnp.zeros((16,), jnp.float32)

        @pl.loop(0, 2)
        def _pass(p):
            q = 2 * c + p
            base = q * QROWS

            @pl.loop(0, 10)
            def _z(i):
                pltpu.sync_copy(
                    stage, acc.at[pl.ds(s * 100000 + i * 10000, 10000)])

            plsc.subcore_barrier()

            @pl.loop(s, F_CHUNKS, step=NSUB)
            def _chunk(j):
                roff = j * 16
                pltpu.sync_copy(frow_hbm.at[pl.ds(roff, 16)], rb)
                pltpu.sync_copy(fcol_hbm.at[pl.ds(roff, 16)], cb)
                pltpu.sync_copy(fval_hbm.at[pl.ds(roff, 16)], vb)

                @pl.loop(0, 16)
                def _row(i):
                    @pl.loop(0, 128, step=16)
                    def _lane(l):
                        r = rb[i, pl.ds(l, 16)]
                        cc = cb[i, pl.ds(l, 16)]
                        rel = r - base
                        ok = (rel >= 0) & (rel < QROWS)
                        rb[i, pl.ds(l, 16)] = jnp.where(
                            ok, rel * FF + cc, QFLAT + cc)

                for t in range(16):
                    pltpu.sync_copy(vb.at[t], acc.at[rb.at[t]], add=True)

            plsc.subcore_barrier()

            @pl.loop(0, 10)
            def _out(i):
                off = s * 100000 + i * 10000
                pltpu.sync_copy(acc.at[pl.ds(off, 10000)], stage2)
                pltpu.sync_copy(stage2,
                                x_hbm.at[pl.ds(q * QFLAT + off, 10000)])

    return k(frow, fcol, fval)


def _mlp(x, w1, b1, w2, b2):
    bm = 2000
    grid = (NN // bm,)

    def body(x_ref, w1_ref, b1_ref, w2_ref, b2_ref, z0_ref, z1_ref):
        h = lax.dot_general(
            x_ref[...].astype(jnp.float32), w1_ref[...],
            (((1,), (0,)), ((), ())),
            precision=lax.Precision.HIGHEST,
            preferred_element_type=jnp.float32)
        h = jnp.maximum(h + b1_ref[...], 0.0)
        z = lax.dot_general(
            h, w2_ref[...], (((1,), (0,)), ((), ())),
            precision=lax.Precision.HIGHEST,
            preferred_element_type=jnp.float32) + b2_ref[...]
        z0_ref[...] = z[:, :HALF].astype(jnp.bfloat16)
        z1_ref[...] = z[:, HALF:].astype(jnp.bfloat16)

    return pl.pallas_call(
        body,
        grid=grid,
        in_specs=[
            pl.BlockSpec((bm, FF), lambda i: (i, 0)),
            pl.BlockSpec((FF, HH), lambda i: (0, 0)),
            pl.BlockSpec((1, HH), lambda i: (0, 0)),
            pl.BlockSpec((HH, LL), lambda i: (0, 0)),
            pl.BlockSpec((1, LL), lambda i: (0, 0)),
        ],
        out_specs=[
            pl.BlockSpec((bm, HALF), lambda i: (i, 0)),
            pl.BlockSpec((bm, HALF), lambda i: (i, 0)),
        ],
        out_shape=[
            jax.ShapeDtypeStruct((NN, HALF), jnp.bfloat16),
            jax.ShapeDtypeStruct((NN, HALF), jnp.bfloat16),
        ],
    )(x, w1, b1, w2, b2)


def _propagate(z0, z1, src, dst, w):
    @functools.partial(
        pl.kernel,
        out_type=(
            jax.ShapeDtypeStruct((NN, HALF), jnp.bfloat16),
            jax.ShapeDtypeStruct((NN, HALF), jnp.bfloat16),
        ),
        mesh=_sc_mesh(),
        compiler_params=_sc_params(),
        scratch_types=[
            pltpu.VMEM((CH,), jnp.int32),
            pltpu.VMEM((CH,), jnp.int32),
            pltpu.VMEM((CH, HALF), jnp.bfloat16),
            pltpu.VMEM((CH,), jnp.int32),
            pltpu.VMEM((CH,), jnp.int32),
            pltpu.VMEM((CH, HALF), jnp.bfloat16),
            pltpu.VMEM((CH, HALF), jnp.bfloat16),
            pltpu.VMEM((CH, HALF), jnp.bfloat16),
            pltpu.VMEM((CH,), jnp.int32),
            pltpu.VMEM((CH,), jnp.int32),
            pltpu.VMEM((BR, HALF), jnp.bfloat16),
            pltpu.VMEM((BR, HALF), jnp.bfloat16),
            pltpu.VMEM((BZ, HALF), jnp.bfloat16),
            pltpu.VMEM_SHARED((NN, HALF), jnp.bfloat16),
            pltpu.VMEM_SHARED((NN, HALF), jnp.bfloat16),
            pltpu.SemaphoreType.DMA,
            pltpu.SemaphoreType.DMA,
            pltpu.SemaphoreType.DMA,
            pltpu.SemaphoreType.DMA,
            pltpu.SemaphoreType.DMA,
            pltpu.SemaphoreType.DMA,
            pltpu.SemaphoreType.DMA,
            pltpu.SemaphoreType.DMA,
        ],
    )
    def k(z0_hbm, z1_hbm, src_hbm, dst_hbm, w_hbm, l0_hbm, l1_hbm,
          sb0, db0, wb0, sb1, db1, wb1, rows0, rows1, dbf0, dbf1,
          ba, bb, zb, acc, loc,
          gs0, gs1, ss0, ss1, ws0, ws1, is0, is1):
        c = lax.axis_index("c")
        s = lax.axis_index("s")
        base = s * KTILE

        @pl.loop(0, BZ)
        def _zzb(r):
            zb[r, :] = jnp.zeros((HALF,), jnp.bfloat16)

        def zero_acc(r0):
            pltpu.sync_copy(zb, acc.at[pl.ds(r0, BZ)])
            pltpu.sync_copy(zb, acc.at[pl.ds(r0 + BZ, BZ)])

        def load_idx(cidx, sbx, dbx, wbx, wsx, isx):
            eoff = cidx * CH
            pltpu.async_copy(w_hbm.at[pl.ds(eoff, CH)], wbx, wsx)
            pltpu.async_copy(src_hbm.at[pl.ds(eoff, CH)], sbx, isx)
            pltpu.async_copy(dst_hbm.at[pl.ds(eoff, CH)], dbx, isx)

        def wait_w(wbx, wsx):
            pltpu.make_async_copy(
                w_hbm.at[pl.ds(0, CH)], wbx, wsx).wait()

        def wait_idx(sbx, dbx, isx):
            pltpu.make_async_copy(
                src_hbm.at[pl.ds(0, CH)], sbx, isx).wait()
            pltpu.make_async_copy(
                dst_hbm.at[pl.ds(0, CH)], dbx, isx).wait()

        def cpidx(srcb, dstb):
            @pl.loop(0, CH, step=16)
            def _c(i):
                dstb[pl.ds(i, 16)] = srcb[pl.ds(i, 16)]

        def fire_gathers(sbx, rowsx, gsx):
            pltpu.async_copy(loc.at[sbx], rowsx, gsx)

        def fire_scatters(rowsx, dbx, ssx):
            pltpu.async_copy(rowsx, acc.at[dbx], ssx, add=True)

        def drain(z_ref, bufx, semx):
            pltpu.make_async_copy(z_ref.at[pl.ds(0, CH)], bufx, semx).wait()

        def scale(rowsx, wbx):
            @pl.loop(0, CH, step=8)
            def _scale(e0):
                for i in range(8):
                    rowsx[e0 + i, :] = rowsx[e0 + i, :] * wbx[e0 + i, :]

        def work(z_ref, l_ref):
            @pl.loop(s, NN // BR, step=NSUB)
            def _pro(kk):
                r0 = kk * BR
                pltpu.sync_copy(z_ref.at[pl.ds(r0, BR)], loc.at[pl.ds(r0, BR)])
                zero_acc(r0)

            plsc.subcore_barrier()

            @pl.loop(0, IT)
            def _iter(it):
                load_idx(base, sb0, db0, wb0, ws0, is0)
                wait_idx(sb0, db0, is0)
                cpidx(db0, dbf0)
                fire_gathers(sb0, rows0, gs0)
                load_idx(base + 1, sb1, db1, wb1, ws1, is1)
                wait_idx(sb1, db1, is1)
                cpidx(db1, dbf1)
                fire_gathers(sb1, rows1, gs1)

                @pl.loop(0, KTILE // 2)
                def _pair(g):
                    c0 = base + 2 * g
                    drain(z_ref, rows0, gs0)
                    wait_w(wb0, ws0)
                    scale(rows0, wb0)
                    fire_scatters(rows0, dbf0, ss0)

                    @pl.when(g < KTILE // 2 - 1)
                    def _pf0():
                        load_idx(c0 + 2, sb0, db0, wb0, ws0, is0)

                    drain(z_ref, rows1, gs1)
                    wait_w(wb1, ws1)
                    scale(rows1, wb1)
                    fire_scatters(rows1, dbf1, ss1)

                    @pl.when(g < KTILE // 2 - 1)
                    def _pf1():
                        load_idx(c0 + 3, sb1, db1, wb1, ws1, is1)

                    @pl.when(g < KTILE // 2 - 1)
                    def _prep():
                        drain(z_ref, rows0, ss0)
                        wait_idx(sb0, db0, is0)
                        cpidx(db0, dbf0)
                        fire_gathers(sb0, rows0, gs0)
                        drain(z_ref, rows1, ss1)
                        wait_idx(sb1, db1, is1)
                        cpidx(db1, dbf1)
                        fire_gathers(sb1, rows1, gs1)

                drain(z_ref, rows0, ss0)
                drain(z_ref, rows1, ss1)

                plsc.subcore_barrier()

                @pl.loop(s, NN // BR, step=NSUB)
                def _blend(kk):
                    r0 = kk * BR
                    pltpu.sync_copy(acc.at[pl.ds(r0, BR)], ba)
                    pltpu.sync_copy(z_ref.at[pl.ds(r0, BR)], bb)

                    @pl.loop(0, BR)
                    def _rowb(r):
                        ba[r, :] = (1.0 - AL) * ba[r, :] + AL * bb[r, :]

                    pltpu.sync_copy(ba, loc.at[pl.ds(r0, BR)])
                    zero_acc(r0)

                    @pl.when(it == IT - 1)
                    def _emit():
                        pltpu.sync_copy(ba, l_ref.at[pl.ds(r0, BR)])

                plsc.subcore_barrier()

        @pl.when(c == 0)
        def _():
            work(z0_hbm, l0_hbm)

        @pl.when(c == 1)
        def _():
            work(z1_hbm, l1_hbm)

    return k(z0, z1, src, dst, w)


def _log_softmax(l0, l1):
    bm = 2000
    grid = (NN // bm,)

    def body(l0_ref, l1_ref, o_ref):
        x = jnp.concatenate(
            [l0_ref[...], l1_ref[...]], axis=1).astype(jnp.float32)
        m = jnp.max(x, axis=1, keepdims=True)
        xm = x - m
        lse = jnp.log(jnp.sum(jnp.exp(xm), axis=1, keepdims=True))
        o_ref[...] = xm - lse

    return pl.pallas_call(
        body,
        grid=grid,
        in_specs=[
            pl.BlockSpec((bm, HALF), lambda i: (i, 0)),
            pl.BlockSpec((bm, HALF), lambda i: (i, 0)),
        ],
        out_specs=pl.BlockSpec((bm, LL), lambda i: (i, 0)),
        out_shape=jax.ShapeDtypeStruct((NN, LL), jnp.float32),
    )(l0, l1)


def kernel(feature_indices, feature_values, edge_indices, edge_weights,
           W1, b1, W2, b2):
    frow = feature_indices[0].astype(jnp.int32)
    fcol = feature_indices[1].astype(jnp.int32)
    fval = feature_values.astype(jnp.float32)
    fpad = F_ROWS * 128 - ZF
    frow = jnp.concatenate([frow, jnp.zeros((fpad,), jnp.int32)]).reshape(
        F_ROWS, 128)
    fcol = jnp.concatenate([fcol, jnp.zeros((fpad,), jnp.int32)]).reshape(
        F_ROWS, 128)
    fval = jnp.concatenate([fval, jnp.zeros((fpad,), jnp.float32)]).reshape(
        F_ROWS, 128)

    src = edge_indices[1].astype(jnp.int32)
    dst = edge_indices[0].astype(jnp.int32)
    wgt = edge_weights.astype(jnp.float32)
    epad = E_ROWS * 128 - EE
    src = jnp.concatenate([src, jnp.zeros((epad,), jnp.int32)])
    dst = jnp.concatenate([dst, jnp.zeros((epad,), jnp.int32)])
    wgt = jnp.concatenate([wgt, jnp.zeros((epad,), jnp.float32)])
    wgt = jnp.broadcast_to(
        wgt.astype(jnp.bfloat16)[:, None], (E_ROWS * 128, HALF))

    x = _densify(frow, fcol, fval).reshape(NN, FF)
    z0, z1 = _mlp(x, W1, b1.reshape(1, HH), W2, b2.reshape(1, LL))
    l0, l1 = _propagate(z0, z1, src, dst, wgt)
    return _log_softmax(l0, l1)


# speedup vs baseline: 15.4642x; 1.0012x over previous
import dataclasses
import functools

import jax
import jax.numpy as jnp
from jax import lax
from jax.experimental import pallas as pl
from jax.experimental.pallas import tpu as pltpu
from jax.experimental.pallas import tpu_sc as plsc

NN = 50000
FF = 128
HH = 128
LL = 64
EE = 800000
ZF = 1000000
AL = 0.1
IT = 10

NSUB = 16
HALF = LL // 2
QROWS = NN // 4
QFLAT = QROWS * FF

F_ROWS = 8192
E_ROWS = 6400
F_CHUNKS = F_ROWS // 16
FKT = F_CHUNKS // NSUB
CH = 320
E_CHUNKS = E_ROWS * 128 // CH
KTILE = E_CHUNKS // NSUB

BR = 200
BZ = 100


def _sc_mesh():
    return plsc.VectorSubcoreMesh(core_axis_name="c", subcore_axis_name="s")


def _sc_params():
    cp = pltpu.CompilerParams()
    if "needs_layout_passes" in pltpu.CompilerParams.__dataclass_fields__:
        cp = dataclasses.replace(cp, needs_layout_passes=False)
    if "use_tc_tiling_on_sc" in pltpu.CompilerParams.__dataclass_fields__:
        cp = dataclasses.replace(cp, use_tc_tiling_on_sc=False)
    return cp


def _densify(frow, fcol, fval):
    @functools.partial(
        pl.kernel,
        out_type=jax.ShapeDtypeStruct((NN * FF,), jnp.float32),
        mesh=_sc_mesh(),
        scratch_types=[
            pltpu.VMEM((16, 128), jnp.int32),
            pltpu.VMEM((16, 128), jnp.int32),
            pltpu.VMEM((16, 128), jnp.float32),
            pltpu.VMEM((16, 128), jnp.int32),
            pltpu.VMEM((16, 128), jnp.int32),
            pltpu.VMEM((16, 128), jnp.float32),
            pltpu.VMEM((5000,), jnp.float32),
            pltpu.VMEM((5000,), jnp.float32),
            pltpu.VMEM_SHARED((QFLAT + 128,), jnp.float32),
            pltpu.SemaphoreType.DMA,
            pltpu.SemaphoreType.DMA,
            pltpu.SemaphoreType.DMA,
            pltpu.SemaphoreType.DMA,
        ],
    )
    def k(frow_hbm, fcol_hbm, fval_hbm, x_hbm,
          rb0, cb0, vb0, rb1, cb1, vb1, stage, stage2, acc,
          ds0, ds1, sc0, sc1):
        c = lax.axis_index("c")
        s = lax.axis_index("s")

        @pl.loop(0, 5000, step=16)
        def _zstage(i):
            stage[pl.ds(i, 16)] = jnp.zeros((16,), jnp.float32)

        def load(j, rbx, cbx, vbx, dsx):
            roff = j * 16
            pltpu.async_copy(frow_hbm.at[pl.ds(roff, 16)], rbx, dsx)
            pltpu.async_copy(fcol_hbm.at[pl.ds(roff, 16)], cbx, dsx)
            pltpu.async_copy(fval_hbm.at[pl.ds(roff, 16)], vbx, dsx)

        def wait_load(rbx, cbx, vbx, dsx):
            pltpu.make_async_copy(
                frow_hbm.at[pl.ds(0, 16)], rbx, dsx).wait()
            pltpu.make_async_copy(
                fcol_hbm.at[pl.ds(0, 16)], cbx, dsx).wait()
            pltpu.make_async_copy(
                fval_hbm.at[pl.ds(0, 16)], vbx, dsx).wait()

        def compute_idx(rbx, cbx, base):
            @pl.loop(0, 16)
            def _row(i):
                @pl.loop(0, 128, step=16)
                def _lane(l):
                    r = rbx[i, pl.ds(l, 16)]
                    cc = cbx[i, pl.ds(l, 16)]
                    rel = r - base
                    ok = (rel >= 0) & (rel < QROWS)
                    rbx[i, pl.ds(l, 16)] = jnp.where(
                        ok, rel * FF + cc, QFLAT + cc)

        def fire_scatter(rbx, vbx, scx):
            for t in range(16):
                pltpu.async_copy(vbx.at[t], acc.at[rbx.at[t]], scx, add=True)

        def drain_scatter(rbx, vbx, scx):
            for t in range(16):
                pltpu.make_async_copy(
                    vbx.at[t], acc.at[rbx.at[t]], scx).wait()

        @pl.loop(0, 2)
        def _pass(p):
            q = 2 * c + p
            base = q * QROWS
            kt = s * FKT

            @pl.loop(0, 20)
            def _z(i):
                pltpu.sync_copy(
                    stage, acc.at[pl.ds(s * 100000 + i * 5000, 5000)])

            plsc.subcore_barrier()

            load(kt, rb0, cb0, vb0, ds0)
            load(kt + 1, rb1, cb1, vb1, ds1)

            @pl.loop(0, FKT // 2)
            def _pairf(g):
                c0 = kt + 2 * g
                wait_load(rb0, cb0, vb0, ds0)
                compute_idx(rb0, cb0, base)
                fire_scatter(rb0, vb0, sc0)
                wait_load(rb1, cb1, vb1, ds1)
                compute_idx(rb1, cb1, base)
                fire_scatter(rb1, vb1, sc1)

                @pl.when(g < FKT // 2 - 1)
                def _prepf():
                    drain_scatter(rb0, vb0, sc0)
                    load(c0 + 2, rb0, cb0, vb0, ds0)
                    drain_scatter(rb1, vb1, sc1)
                    load(c0 + 3, rb1, cb1, vb1, ds1)

            drain_scatter(rb0, vb0, sc0)
            drain_scatter(rb1, vb1, sc1)

            plsc.subcore_barrier()

            @pl.loop(0, 20)
            def _out(i):
                off = s * 100000 + i * 5000
                pltpu.sync_copy(acc.at[pl.ds(off, 5000)], stage2)
                pltpu.sync_copy(stage2,
                                x_hbm.at[pl.ds(q * QFLAT + off, 5000)])

    return k(frow, fcol, fval)


def _mlp(x, w1, b1, w2, b2):
    bm = 2000
    grid = (NN // bm,)

    def body(x_ref, w1_ref, b1_ref, w2_ref, b2_ref, z0_ref, z1_ref):
        h = lax.dot_general(
            x_ref[...].astype(jnp.float32), w1_ref[...],
            (((1,), (0,)), ((), ())),
            precision=lax.Precision.HIGHEST,
            preferred_element_type=jnp.float32)
        h = jnp.maximum(h + b1_ref[...], 0.0)
        z = lax.dot_general(
            h, w2_ref[...], (((1,), (0,)), ((), ())),
            precision=lax.Precision.HIGHEST,
            preferred_element_type=jnp.float32) + b2_ref[...]
        z0_ref[...] = z[:, :HALF].astype(jnp.bfloat16)
        z1_ref[...] = z[:, HALF:].astype(jnp.bfloat16)

    return pl.pallas_call(
        body,
        grid=grid,
        in_specs=[
            pl.BlockSpec((bm, FF), lambda i: (i, 0)),
            pl.BlockSpec((FF, HH), lambda i: (0, 0)),
            pl.BlockSpec((1, HH), lambda i: (0, 0)),
            pl.BlockSpec((HH, LL), lambda i: (0, 0)),
            pl.BlockSpec((1, LL), lambda i: (0, 0)),
        ],
        out_specs=[
            pl.BlockSpec((bm, HALF), lambda i: (i, 0)),
            pl.BlockSpec((bm, HALF), lambda i: (i, 0)),
        ],
        out_shape=[
            jax.ShapeDtypeStruct((NN, HALF), jnp.bfloat16),
            jax.ShapeDtypeStruct((NN, HALF), jnp.bfloat16),
        ],
    )(x, w1, b1, w2, b2)


def _propagate(z0, z1, src, dst, w):
    @functools.partial(
        pl.kernel,
        out_type=(
            jax.ShapeDtypeStruct((NN, HALF), jnp.bfloat16),
            jax.ShapeDtypeStruct((NN, HALF), jnp.bfloat16),
        ),
        mesh=_sc_mesh(),
        compiler_params=_sc_params(),
        scratch_types=[
            pltpu.VMEM((CH,), jnp.int32),
            pltpu.VMEM((CH,), jnp.int32),
            pltpu.VMEM((CH, HALF), jnp.bfloat16),
            pltpu.VMEM((CH,), jnp.int32),
            pltpu.VMEM((CH,), jnp.int32),
            pltpu.VMEM((CH, HALF), jnp.bfloat16),
            pltpu.VMEM((CH, HALF), jnp.bfloat16),
            pltpu.VMEM((CH, HALF), jnp.bfloat16),
            pltpu.VMEM((CH,), jnp.int32),
            pltpu.VMEM((CH,), jnp.int32),
            pltpu.VMEM((BR, HALF), jnp.bfloat16),
            pltpu.VMEM((BR, HALF), jnp.bfloat16),
            pltpu.VMEM((BZ, HALF), jnp.bfloat16),
            pltpu.VMEM_SHARED((NN, HALF), jnp.bfloat16),
            pltpu.VMEM_SHARED((NN, HALF), jnp.bfloat16),
            pltpu.SemaphoreType.DMA,
            pltpu.SemaphoreType.DMA,
            pltpu.SemaphoreType.DMA,
            pltpu.SemaphoreType.DMA,
            pltpu.SemaphoreType.DMA,
            pltpu.SemaphoreType.DMA,
            pltpu.SemaphoreType.DMA,
            pltpu.SemaphoreType.DMA,
        ],
    )
    def k(z0_hbm, z1_hbm, src_hbm, dst_hbm, w_hbm, l0_hbm, l1_hbm,
          sb0, db0, wb0, sb1, db1, wb1, rows0, rows1, dbf0, dbf1,
          ba, bb, zb, acc, loc,
          gs0, gs1, ss0, ss1, ws0, ws1, is0, is1):
        c = lax.axis_index("c")
        s = lax.axis_index("s")
        base = s * KTILE

        @pl.loop(0, BZ)
        def _zzb(r):
            zb[r, :] = jnp.zeros((HALF,), jnp.bfloat16)

        def zero_acc(r0):
            pltpu.sync_copy(zb, acc.at[pl.ds(r0, BZ)])
            pltpu.sync_copy(zb, acc.at[pl.ds(r0 + BZ, BZ)])

        def load_idx(cidx, sbx, dbx, wbx, wsx, isx):
            eoff = cidx * CH
            pltpu.async_copy(w_hbm.at[pl.ds(eoff, CH)], wbx, wsx)
            pltpu.async_copy(src_hbm.at[pl.ds(eoff, CH)], sbx, isx)
            pltpu.async_copy(dst_hbm.at[pl.ds(eoff, CH)], dbx, isx)

        def wait_w(wbx, wsx):
            pltpu.make_async_copy(
                w_hbm.at[pl.ds(0, CH)], wbx, wsx).wait()

        def wait_idx(sbx, dbx, isx):
            pltpu.make_async_copy(
                src_hbm.at[pl.ds(0, CH)], sbx, isx).wait()
            pltpu.make_async_copy(
                dst_hbm.at[pl.ds(0, CH)], dbx, isx).wait()

        def cpidx(srcb, dstb):
            @pl.loop(0, CH, step=16)
            def _c(i):
                dstb[pl.ds(i, 16)] = srcb[pl.ds(i, 16)]

        def fire_gathers(sbx, rowsx, gsx):
            pltpu.async_copy(loc.at[sbx], rowsx, gsx)

        def fire_scatters(rowsx, dbx, ssx):
            pltpu.async_copy(rowsx, acc.at[dbx], ssx, add=True)

        def drain(z_ref, bufx, semx):
            pltpu.make_async_copy(z_ref.at[pl.ds(0, CH)], bufx, semx).wait()

        def scale(rowsx, wbx):
            @pl.loop(0, CH, step=8)
            def _scale(e0):
                for i in range(8):
                    rowsx[e0 + i, :] = rowsx[e0 + i, :] * wbx[e0 + i, :]

        def work(z_ref, l_ref):
            @pl.loop(s, NN // BR, step=NSUB)
            def _pro(kk):
                r0 = kk * BR
                pltpu.sync_copy(z_ref.at[pl.ds(r0, BR)], loc.at[pl.ds(r0, BR)])
                zero_acc(r0)

            plsc.subcore_barrier()

            @pl.loop(0, IT)
            def _iter(it):
                load_idx(base, sb0, db0, wb0, ws0, is0)
                wait_idx(sb0, db0, is0)
                cpidx(db0, dbf0)
                fire_gathers(sb0, rows0, gs0)
                load_idx(base + 1, sb1, db1, wb1, ws1, is1)
                wait_idx(sb1, db1, is1)
                cpidx(db1, dbf1)
                fire_gathers(sb1, rows1, gs1)

                @pl.loop(0, KTILE // 2)
                def _pair(g):
                    c0 = base + 2 * g
                    drain(z_ref, rows0, gs0)
                    wait_w(wb0, ws0)
                    scale(rows0, wb0)
                    fire_scatters(rows0, dbf0, ss0)

                    @pl.when(g < KTILE // 2 - 1)
                    def _pf0():
                        load_idx(c0 + 2, sb0, db0, wb0, ws0, is0)

                    drain(z_ref, rows1, gs1)
                    wait_w(wb1, ws1)
                    scale(rows1, wb1)
                    fire_scatters(rows1, dbf1, ss1)

                    @pl.when(g < KTILE // 2 - 1)
                    def _pf1():
                        load_idx(c0 + 3, sb1, db1, wb1, ws1, is1)

                    @pl.when(g < KTILE // 2 - 1)
                    def _prep():
                        drain(z_ref, rows0, ss0)
                        wait_idx(sb0, db0, is0)
                        cpidx(db0, dbf0)
                        fire_gathers(sb0, rows0, gs0)
                        drain(z_ref, rows1, ss1)
                        wait_idx(sb1, db1, is1)
                        cpidx(db1, dbf1)
                        fire_gathers(sb1, rows1, gs1)

                drain(z_ref, rows0, ss0)
                drain(z_ref, rows1, ss1)

                plsc.subcore_barrier()

                @pl.loop(s, NN // BR, step=NSUB)
                def _blend(kk):
                    r0 = kk * BR
                    pltpu.sync_copy(acc.at[pl.ds(r0, BR)], ba)
                    pltpu.sync_copy(z_ref.at[pl.ds(r0, BR)], bb)

                    @pl.loop(0, BR)
                    def _rowb(r):
                        ba[r, :] = (1.0 - AL) * ba[r, :] + AL * bb[r, :]

                    pltpu.sync_copy(ba, loc.at[pl.ds(r0, BR)])
                    zero_acc(r0)

                    @pl.when(it == IT - 1)
                    def _emit():
                        pltpu.sync_copy(ba, l_ref.at[pl.ds(r0, BR)])

                plsc.subcore_barrier()

        @pl.when(c == 0)
        def _():
            work(z0_hbm, l0_hbm)

        @pl.when(c == 1)
        def _():
            work(z1_hbm, l1_hbm)

    return k(z0, z1, src, dst, w)


def _log_softmax(l0, l1):
    bm = 2000
    grid = (NN // bm,)

    def body(l0_ref, l1_ref, o_ref):
        x = jnp.concatenate(
            [l0_ref[...], l1_ref[...]], axis=1).astype(jnp.float32)
        m = jnp.max(x, axis=1, keepdims=True)
        xm = x - m
        lse = jnp.log(jnp.sum(jnp.exp(xm), axis=1, keepdims=True))
        o_ref[...] = xm - lse

    return pl.pallas_call(
        body,
        grid=grid,
        in_specs=[
            pl.BlockSpec((bm, HALF), lambda i: (i, 0)),
            pl.BlockSpec((bm, HALF), lambda i: (i, 0)),
        ],
        out_specs=pl.BlockSpec((bm, LL), lambda i: (i, 0)),
        out_shape=jax.ShapeDtypeStruct((NN, LL), jnp.float32),
    )(l0, l1)


def kernel(feature_indices, feature_values, edge_indices, edge_weights,
           W1, b1, W2, b2):
    frow = feature_indices[0].astype(jnp.int32)
    fcol = feature_indices[1].astype(jnp.int32)
    fval = feature_values.astype(jnp.float32)
    fpad = F_ROWS * 128 - ZF
    frow = jnp.concatenate([frow, jnp.zeros((fpad,), jnp.int32)]).reshape(
        F_ROWS, 128)
    fcol = jnp.concatenate([fcol, jnp.zeros((fpad,), jnp.int32)]).reshape(
        F_ROWS, 128)
    fval = jnp.concatenate([fval, jnp.zeros((fpad,), jnp.float32)]).reshape(
        F_ROWS, 128)

    src = edge_indices[1].astype(jnp.int32)
    dst = edge_indices[0].astype(jnp.int32)
    wgt = edge_weights.astype(jnp.float32)
    epad = E_ROWS * 128 - EE
    src = jnp.concatenate([src, jnp.zeros((epad,), jnp.int32)])
    dst = jnp.concatenate([dst, jnp.zeros((epad,), jnp.int32)])
    wgt = jnp.concatenate([wgt, jnp.zeros((epad,), jnp.float32)])
    wgt = jnp.broadcast_to(
        wgt.astype(jnp.bfloat16)[:, None], (E_ROWS * 128, HALF))

    x = _densify(frow, fcol, fval).reshape(NN, FF)
    z0, z1 = _mlp(x, W1, b1.reshape(1, HH), W2, b2.reshape(1, LL))
    l0, l1 = _propagate(z0, z1, src, dst, wgt)
    return _log_softmax(l0, l1)
